# baseline (speedup 1.0000x reference)
import functools
import math

import jax
import jax.numpy as jnp
from jax import lax
from jax.experimental import pallas as pl
from jax.experimental.pallas import tpu as pltpu

B, N, K, DIM, HEAD_DIM = 4, 2048, 24, 256, 256
EPS = 1e-5
BN_SCALE = 1.0 / math.sqrt(1.0 + EPS)
ROWS = B * N


def _gelu(x):
    return 0.5 * x * (1.0 + lax.erf(x * (1.0 / math.sqrt(2.0))))


def _nbr_mlp_kernel(nbr_ref, w1_ref, b1_ref, w2_ref, b2_ref, w3_ref, out_ref):
    j = pl.program_id(1)
    x = nbr_ref[0]
    h = _gelu(jnp.dot(x, w1_ref[...], preferred_element_type=jnp.float32)
              * BN_SCALE + b1_ref[...])
    h = _gelu(jnp.dot(h, w2_ref[...], preferred_element_type=jnp.float32)
              * BN_SCALE + b2_ref[...])
    h = jnp.dot(h, w3_ref[...], preferred_element_type=jnp.float32)

    @pl.when(j == 0)
    def _():
        out_ref[...] = h

    @pl.when(j > 0)
    def _():
        out_ref[...] = jnp.maximum(out_ref[...], h)


def _nbr_mlp(nbr, w1, b1, w2, b2, w3, p_blk=512):
    grid = (ROWS // p_blk, K)
    return pl.pallas_call(
        _nbr_mlp_kernel,
        grid=grid,
        in_specs=[
            pl.BlockSpec((1, p_blk, 8), lambda i, j: (j, i, 0)),
            pl.BlockSpec((8, 16), lambda i, j: (0, 0)),
            pl.BlockSpec((1, 16), lambda i, j: (0, 0)),
            pl.BlockSpec((16, 32), lambda i, j: (0, 0)),
            pl.BlockSpec((1, 32), lambda i, j: (0, 0)),
            pl.BlockSpec((32, DIM), lambda i, j: (0, 0)),
        ],
        out_specs=pl.BlockSpec((p_blk, DIM), lambda i, j: (i, 0)),
        out_shape=jax.ShapeDtypeStruct((ROWS, DIM), jnp.float32),
    )(nbr, w1, b1, w2, b2, w3)


def _bn_mlp_kernel(h_ref, g0_ref, b0_ref, w1_ref, b1_ref, w2_ref, g_ref,
                   b_ref, out_ref):
    xx = h_ref[...] * BN_SCALE * g0_ref[...] + b0_ref[...]
    t = _gelu(jnp.dot(xx, w1_ref[...], preferred_element_type=jnp.float32)
              + b1_ref[...])
    t = jnp.dot(t, w2_ref[...], preferred_element_type=jnp.float32)
    out_ref[...] = xx + t * BN_SCALE * g_ref[...] + b_ref[...]


def _bn_mlp(h, g0, b0, w1, b1, w2, g, b, p_blk=1024):
    return pl.pallas_call(
        _bn_mlp_kernel,
        grid=(ROWS // p_blk,),
        in_specs=[
            pl.BlockSpec((p_blk, DIM), lambda i: (i, 0)),
            pl.BlockSpec((1, DIM), lambda i: (0, 0)),
            pl.BlockSpec((1, DIM), lambda i: (0, 0)),
            pl.BlockSpec((DIM, 2 * DIM), lambda i: (0, 0)),
            pl.BlockSpec((1, 2 * DIM), lambda i: (0, 0)),
            pl.BlockSpec((2 * DIM, DIM), lambda i: (0, 0)),
            pl.BlockSpec((1, DIM), lambda i: (0, 0)),
            pl.BlockSpec((1, DIM), lambda i: (0, 0)),
        ],
        out_specs=pl.BlockSpec((p_blk, DIM), lambda i: (i, 0)),
        out_shape=jax.ShapeDtypeStruct((ROWS, DIM), jnp.float32),
    )(h, g0, b0, w1, b1, w2, g, b)


def _matmul_kernel(x_ref, w_ref, out_ref):
    out_ref[...] = jnp.dot(x_ref[...], w_ref[...],
                           preferred_element_type=jnp.float32)


def _matmul(x, w, p_blk=1024):
    m, k = x.shape
    _, n = w.shape
    return pl.pallas_call(
        _matmul_kernel,
        grid=(m // p_blk,),
        in_specs=[
            pl.BlockSpec((p_blk, k), lambda i: (i, 0)),
            pl.BlockSpec((k, n), lambda i: (0, 0)),
        ],
        out_specs=pl.BlockSpec((p_blk, n), lambda i: (i, 0)),
        out_shape=jax.ShapeDtypeStruct((m, n), jnp.float32),
    )(x, w)


def _combine_kernel(xx_ref, gmax_ref, y_ref, g_ref, b_ref, out_ref):
    edge = gmax_ref[...] - y_ref[...]
    out_ref[...] = xx_ref[...] + edge * BN_SCALE * g_ref[...] + b_ref[...]


def _combine(xx, gmax, y, g, b, p_blk=1024):
    return pl.pallas_call(
        _combine_kernel,
        grid=(ROWS // p_blk,),
        in_specs=[
            pl.BlockSpec((p_blk, DIM), lambda i: (i, 0)),
            pl.BlockSpec((p_blk, DIM), lambda i: (i, 0)),
            pl.BlockSpec((p_blk, DIM), lambda i: (i, 0)),
            pl.BlockSpec((1, DIM), lambda i: (0, 0)),
            pl.BlockSpec((1, DIM), lambda i: (0, 0)),
        ],
        out_specs=pl.BlockSpec((p_blk, DIM), lambda i: (i, 0)),
        out_shape=jax.ShapeDtypeStruct((ROWS, DIM), jnp.float32),
    )(xx, gmax, y, g, b)


def _res_mlp_kernel(xx_ref, w1_ref, b1_ref, w2_ref, g_ref, b_ref, out_ref):
    xx = xx_ref[...]
    t = _gelu(jnp.dot(xx, w1_ref[...], preferred_element_type=jnp.float32)
              + b1_ref[...])
    t = jnp.dot(t, w2_ref[...], preferred_element_type=jnp.float32)
    out_ref[...] = xx + t * BN_SCALE * g_ref[...] + b_ref[...]


def _res_mlp(xx, w1, b1, w2, g, b, p_blk=1024):
    return pl.pallas_call(
        _res_mlp_kernel,
        grid=(ROWS // p_blk,),
        in_specs=[
            pl.BlockSpec((p_blk, DIM), lambda i: (i, 0)),
            pl.BlockSpec((DIM, 2 * DIM), lambda i: (0, 0)),
            pl.BlockSpec((1, 2 * DIM), lambda i: (0, 0)),
            pl.BlockSpec((2 * DIM, DIM), lambda i: (0, 0)),
            pl.BlockSpec((1, DIM), lambda i: (0, 0)),
            pl.BlockSpec((1, DIM), lambda i: (0, 0)),
        ],
        out_specs=pl.BlockSpec((p_blk, DIM), lambda i: (i, 0)),
        out_shape=jax.ShapeDtypeStruct((ROWS, DIM), jnp.float32),
    )(xx, w1, b1, w2, g, b)


def _final_kernel(xx_ref, g_ref, b_ref, w_ref, out_ref):
    t = xx_ref[...] * BN_SCALE * g_ref[...] + b_ref[...]
    out_ref[...] = jnp.dot(t, w_ref[...], preferred_element_type=jnp.float32)


def _final(xx, g, b, w, p_blk=1024):
    return pl.pallas_call(
        _final_kernel,
        grid=(ROWS // p_blk,),
        in_specs=[
            pl.BlockSpec((p_blk, DIM), lambda i: (i, 0)),
            pl.BlockSpec((1, DIM), lambda i: (0, 0)),
            pl.BlockSpec((1, DIM), lambda i: (0, 0)),
            pl.BlockSpec((DIM, HEAD_DIM), lambda i: (0, 0)),
        ],
        out_specs=pl.BlockSpec((p_blk, HEAD_DIM), lambda i: (i, 0)),
        out_shape=jax.ShapeDtypeStruct((ROWS, HEAD_DIM), jnp.float32),
    )(xx, g, b, w)


def _gather_nbrs(x, idx):
    Bb, Nn, k = idx.shape
    C = x.shape[-1]
    return jnp.take_along_axis(
        x, idx.reshape(Bb, Nn * k, 1), axis=1).reshape(Bb, Nn, k, C)


def kernel(x, xyz, prev_knn, pwd, ne_w1, ne_g1, ne_b1, ne_w2, ne_g2, ne_b2,
           ne_w3, nbr_g, nbr_b, m_w1, m_b1, m_w2, m_g, m_b, lfp_w, lfp_g,
           lfp_b, mm_w1, mm_b1, mm_w2, mm_g, mm_b, pp_g, pp_b, pp_w):
    _, knn = lax.top_k(-pwd, K)

    nbr_xyz = _gather_nbrs(xyz, knn) - xyz[:, :, None, :]
    height = xyz[..., 1:2] / 10.0
    height = height - height.min(axis=1, keepdims=True)
    feat = jnp.concatenate([x, height], axis=-1)
    nbr_feat = _gather_nbrs(feat, knn)
    nbr = jnp.concatenate(
        [nbr_xyz, nbr_feat,
         jnp.zeros_like(nbr_xyz[..., :1])], axis=-1).reshape(ROWS, K, 8)
    nbr = jnp.transpose(nbr, (1, 0, 2))

    w1p = jnp.pad(ne_w1, ((0, 1), (0, 0)))
    h = _nbr_mlp(nbr, w1p, ne_b1[None], ne_w2, ne_b2[None], ne_w3)

    xx = _bn_mlp(h, nbr_g[None], nbr_b[None], m_w1, m_b1[None], m_w2, m_g[None], m_b[None])

    for i in range(4):
        y = _matmul(xx, lfp_w[i])
        y3 = y.reshape(B, N, DIM)
        gmax = _gather_nbrs(y3, knn).max(axis=2).reshape(ROWS, DIM)
        xx = _combine(xx, gmax, y, lfp_g[i][None], lfp_b[i][None])
        if i % 2 == 1:
            j = i // 2
            xx = _res_mlp(xx, mm_w1[j], mm_b1[j][None], mm_w2[j], mm_g[j][None], mm_b[j][None])

    return _final(xx, pp_g[None], pp_b[None], pp_w)


# speedup vs baseline: 13.0895x; 13.0895x over previous
import functools
import math

import jax
import jax.numpy as jnp
from jax import lax
from jax.experimental import pallas as pl
from jax.experimental.pallas import tpu as pltpu
from jax.experimental.pallas import tpu_sc as plsc

B, N, K, DIM, HEAD_DIM = 4, 2048, 24, 256, 256
EPS = 1e-5
BN_SCALE = 1.0 / math.sqrt(1.0 + EPS)
ROWS = B * N

NC, NS = 2, 16
NW = NC * NS
PPW = ROWS // NW
PG = 4
NG = PPW // PG
GR = PG * K


def _gmax_body(tab_hbm, idx_hbm, out_hbm, idx_v, rows0, rows1, out_v, sem0,
               sem1):
    wid = lax.axis_index("s") * NC + lax.axis_index("c")
    pltpu.sync_copy(idx_hbm.at[wid], idx_v)
    bufs = (rows0, rows1)
    sems = (sem0, sem1)

    def issue(g, slot):
        pltpu.async_copy(tab_hbm.at[idx_v.at[g]], bufs[slot], sems[slot])

    def wait(slot):
        pltpu.make_async_copy(tab_hbm.at[pl.ds(0, GR)], bufs[slot],
                              sems[slot]).wait()

    def compute(g, slot):
        rows = bufs[slot]

        def p_body(p, _):
            r0 = p * K

            def t_body(t, _):
                c0 = pl.multiple_of(t * 16, 16)
                acc = rows[r0, pl.ds(c0, 16)]
                for j in range(1, K):
                    acc = jnp.maximum(acc, rows[r0 + j, pl.ds(c0, 16)])
                out_v[g * PG + p, pl.ds(c0, 16)] = acc
                return 0

            lax.fori_loop(0, DIM // 16, t_body, 0)
            return 0

        lax.fori_loop(0, PG, p_body, 0)

    issue(0, 0)

    def g_body(g2, _):
        g = g2 * 2
        wait(0)
        issue(g + 1, 1)
        compute(g, 0)
        wait(1)

        @pl.when(g + 2 < NG)
        def _():
            issue(g + 2, 0)

        compute(g + 1, 1)
        return 0

    lax.fori_loop(0, NG // 2, g_body, 0)
    pltpu.sync_copy(out_v, out_hbm.at[pl.ds(wid * PPW, PPW)])


def _sc_gather_max(tab, idx3):
    mesh = plsc.VectorSubcoreMesh(core_axis_name="c", subcore_axis_name="s")
    f = pl.kernel(
        _gmax_body,
        mesh=mesh,
        out_type=jax.ShapeDtypeStruct((ROWS, DIM), jnp.float32),
        scratch_types=[
            pltpu.VMEM((NG, GR), jnp.int32),
            pltpu.VMEM((GR, DIM), jnp.float32),
            pltpu.VMEM((GR, DIM), jnp.float32),
            pltpu.VMEM((PPW, DIM), jnp.float32),
            pltpu.SemaphoreType.DMA,
            pltpu.SemaphoreType.DMA,
        ],
        compiler_params=pltpu.CompilerParams(needs_layout_passes=False),
    )
    return f(tab, idx3)


_TKW = 8
_INF = float('inf')


def _merge16_kv(ka, va, kb, vb):
    krb = lax.rev(kb, (0,))
    vrb = lax.rev(vb, (0,))
    cmp = ka <= krb
    lok = jnp.minimum(ka, krb)
    hik = jnp.maximum(ka, krb)
    lov = jnp.where(cmp, va, vrb)
    hiv = jnp.where(cmp, vrb, va)
    lok, lov = plsc.sort_key_val(lok, lov)
    hik, hiv = plsc.sort_key_val(hik, hiv)
    return lok, lov, hik, hiv


def _topk_body(pwd_hbm, out_hbm, buf0, buf1, cval, cidx, outv, sem0, sem1):
    wid = lax.axis_index("s") * NC + lax.axis_index("c")
    row0 = wid * PPW
    boff = (wid // (NW // B)) * N
    ii = jnp.arange(16, dtype=jnp.int32)
    iif = ii.astype(jnp.float32)
    bufs = (buf0, buf1)
    sems = (sem0, sem1)
    nwin = PPW // _TKW

    def issue(g, slot):
        pltpu.async_copy(pwd_hbm.at[pl.ds(row0 + g * _TKW, _TKW)],
                         bufs[slot], sems[slot])

    def wait(slot):
        pltpu.make_async_copy(pwd_hbm.at[pl.ds(0, _TKW)], bufs[slot],
                              sems[slot]).wait()

    def window(g, slot):
        buf = bufs[slot]

        def row_body(r, _):
            def pa(c, carry):
                m1, m2 = carry
                v = buf[r, pl.ds(pl.multiple_of(c * 16, 16), 16)]
                nm1 = jnp.minimum(m1, v)
                m2 = jnp.minimum(m2, jnp.maximum(m1, v))
                return nm1, m2

            m1, m2 = lax.fori_loop(
                0, 128, pa, (jnp.full((16,), _INF), jnp.full((16,), _INF)))
            s1, _ = plsc.sort_key_val(m1, ii)
            s2, _ = plsc.sort_key_val(m2, ii)
            _, _, hik, _ = _merge16_kv(s1, iif, s2, iif)
            tau = jnp.max(jnp.where(ii == 7, hik, -_INF))
            tauv = jnp.full((16,), tau)

            def pb(c, off):
                cb = pl.multiple_of(c * 16, 16)
                v = buf[r, pl.ds(cb, 16)]
                msk = v <= tauv
                mi = msk.astype(jnp.int32)
                nm = jnp.max(plsc.cumsum(mi))

                def hit(off):
                    pos = off + plsc.cumsum(mi) - 1
                    plsc.store_scatter(cval, [pos], v, mask=msk)
                    plsc.store_scatter(cidx, [pos], ii + c * 16, mask=msk)
                    return off + nm

                return lax.cond(nm > 0, hit, lambda off: off, off)

            cnt = lax.fori_loop(0, 128, pb, jnp.int32(0))

            nchunk = (cnt + 15) // 16
            rem0 = cnt - (nchunk - 1) * 16

            def sel(ci, carry):
                r0k, r0v, r1k, r1v = carry
                cb = ci * 16
                ck = plsc.load_gather(cval, [cb + ii])
                cv = plsc.load_gather(cidx, [cb + ii])
                nvalid = jnp.where(ci == nchunk - 1, rem0, 16)
                ck = jnp.where(ii < nvalid, ck, _INF)
                ck, cv = plsc.sort_key_val(ck, cv)
                crk = lax.rev(ck, (0,))
                crv = lax.rev(cv, (0,))
                cmp = r1k <= crk
                n1k = jnp.minimum(r1k, crk)
                n1v = jnp.where(cmp, r1v, crv)
                n1k, n1v = plsc.sort_key_val(n1k, n1v)
                return _merge16_kv(r0k, r0v, n1k, n1v)

            init = (jnp.full((16,), _INF), ii, jnp.full((16,), _INF), ii)
            r0k, r0v, r1k, r1v = lax.fori_loop(0, nchunk, sel, init)

            p = g * _TKW + r
            outv[p, pl.ds(0, 16)] = r0v + boff
            plsc.store_scatter(outv.at[p], [ii + 16], r1v + boff, mask=ii < 8)
            return 0

        lax.fori_loop(0, _TKW, row_body, 0)

    issue(0, 0)

    def w_body(g2, _):
        g = g2 * 2
        wait(0)
        issue(g + 1, 1)
        window(g, 0)
        wait(1)

        @pl.when(g + 2 < nwin)
        def _():
            issue(g + 2, 0)

        window(g + 1, 1)
        return 0

    lax.fori_loop(0, nwin // 2, w_body, 0)
    pltpu.sync_copy(outv, out_hbm.at[pl.ds(row0, PPW)])


def _sc_topk(pwd2):
    mesh = plsc.VectorSubcoreMesh(core_axis_name="c", subcore_axis_name="s")
    f = pl.kernel(
        _topk_body,
        mesh=mesh,
        out_type=jax.ShapeDtypeStruct((ROWS, K), jnp.int32),
        scratch_types=[
            pltpu.VMEM((_TKW, N), jnp.float32),
            pltpu.VMEM((_TKW, N), jnp.float32),
            pltpu.VMEM((N,), jnp.float32),
            pltpu.VMEM((N,), jnp.int32),
            pltpu.VMEM((PPW, K), jnp.int32),
            pltpu.SemaphoreType.DMA,
            pltpu.SemaphoreType.DMA,
        ],
        compiler_params=pltpu.CompilerParams(needs_layout_passes=False),
    )
    return f(pwd2)


def _g8_body(tab_hbm, idx_hbm, out_hbm, tab_v, idx_v, out_v, sem):
    wid = lax.axis_index("s") * NC + lax.axis_index("c")
    pltpu.sync_copy(tab_hbm, tab_v)
    pltpu.sync_copy(idx_hbm.at[wid], idx_v)
    ii = jnp.arange(16, dtype=jnp.int32)

    def q_body(q, _):
        a0 = plsc.load_gather(idx_v, [q * 16 + ii]) * 8
        o0 = q * 128 + ii * 8
        for c in range(8):
            g = plsc.load_gather(tab_v, [a0 + c])
            plsc.store_scatter(out_v, [o0 + c], g)
        return 0

    lax.fori_loop(0, (PPW * K) // 16, q_body, 0)
    pltpu.sync_copy(out_v, out_hbm.at[pl.ds(wid * PPW * K * 8, PPW * K * 8)])


def _sc_gather8(tab8, idx2):
    mesh = plsc.VectorSubcoreMesh(core_axis_name="c", subcore_axis_name="s")
    f = pl.kernel(
        _g8_body,
        mesh=mesh,
        out_type=jax.ShapeDtypeStruct((ROWS * K * 8,), jnp.float32),
        scratch_types=[
            pltpu.VMEM((ROWS * 8,), jnp.float32),
            pltpu.VMEM((PPW * K,), jnp.int32),
            pltpu.VMEM((PPW * K * 8,), jnp.float32),
            pltpu.SemaphoreType.DMA,
        ],
        compiler_params=pltpu.CompilerParams(needs_layout_passes=False),
    )
    return f(tab8, idx2)


def _gelu(x):
    return 0.5 * x * (1.0 + lax.erf(x * (1.0 / math.sqrt(2.0))))


def _nbr_mlp_kernel(nbr_ref, c8_ref, w1_ref, b1_ref, w2_ref, b2_ref, w3_ref,
                    out_ref):
    c8 = c8_ref[...]
    hmax = None
    for j in range(K):
        x = nbr_ref[:, j * 8:(j + 1) * 8] - c8
        h = _gelu(jnp.dot(x, w1_ref[...], preferred_element_type=jnp.float32)
                  * BN_SCALE + b1_ref[...])
        h = _gelu(jnp.dot(h, w2_ref[...], preferred_element_type=jnp.float32)
                  * BN_SCALE + b2_ref[...])
        h = jnp.dot(h, w3_ref[...], preferred_element_type=jnp.float32)
        hmax = h if hmax is None else jnp.maximum(hmax, h)
    out_ref[...] = hmax


def _nbr_mlp(nbr, c8, w1, b1, w2, b2, w3, p_blk=512):
    return pl.pallas_call(
        _nbr_mlp_kernel,
        grid=(ROWS // p_blk,),
        in_specs=[
            pl.BlockSpec((p_blk, K * 8), lambda i: (i, 0)),
            pl.BlockSpec((p_blk, 8), lambda i: (i, 0)),
            pl.BlockSpec((8, 16), lambda i: (0, 0)),
            pl.BlockSpec((1, 16), lambda i: (0, 0)),
            pl.BlockSpec((16, 32), lambda i: (0, 0)),
            pl.BlockSpec((1, 32), lambda i: (0, 0)),
            pl.BlockSpec((32, DIM), lambda i: (0, 0)),
        ],
        out_specs=pl.BlockSpec((p_blk, DIM), lambda i: (i, 0)),
        out_shape=jax.ShapeDtypeStruct((ROWS, DIM), jnp.float32),
    )(nbr, c8, w1, b1, w2, b2, w3)


def _bn_mlp_kernel(h_ref, g0_ref, b0_ref, w1_ref, b1_ref, w2_ref, g_ref,
                   b_ref, out_ref):
    xx = h_ref[...] * BN_SCALE * g0_ref[...] + b0_ref[...]
    t = _gelu(jnp.dot(xx, w1_ref[...], preferred_element_type=jnp.float32)
              + b1_ref[...])
    t = jnp.dot(t, w2_ref[...], preferred_element_type=jnp.float32)
    out_ref[...] = xx + t * BN_SCALE * g_ref[...] + b_ref[...]


def _bn_mlp(h, g0, b0, w1, b1, w2, g, b, p_blk=1024):
    return pl.pallas_call(
        _bn_mlp_kernel,
        grid=(ROWS // p_blk,),
        in_specs=[
            pl.BlockSpec((p_blk, DIM), lambda i: (i, 0)),
            pl.BlockSpec((1, DIM), lambda i: (0, 0)),
            pl.BlockSpec((1, DIM), lambda i: (0, 0)),
            pl.BlockSpec((DIM, 2 * DIM), lambda i: (0, 0)),
            pl.BlockSpec((1, 2 * DIM), lambda i: (0, 0)),
            pl.BlockSpec((2 * DIM, DIM), lambda i: (0, 0)),
            pl.BlockSpec((1, DIM), lambda i: (0, 0)),
            pl.BlockSpec((1, DIM), lambda i: (0, 0)),
        ],
        out_specs=pl.BlockSpec((p_blk, DIM), lambda i: (i, 0)),
        out_shape=jax.ShapeDtypeStruct((ROWS, DIM), jnp.float32),
    )(h, g0, b0, w1, b1, w2, g, b)


def _matmul_kernel(x_ref, w_ref, out_ref):
    out_ref[...] = jnp.dot(x_ref[...], w_ref[...],
                           preferred_element_type=jnp.float32)


def _matmul(x, w, p_blk=1024):
    m, k = x.shape
    _, n = w.shape
    return pl.pallas_call(
        _matmul_kernel,
        grid=(m // p_blk,),
        in_specs=[
            pl.BlockSpec((p_blk, k), lambda i: (i, 0)),
            pl.BlockSpec((k, n), lambda i: (0, 0)),
        ],
        out_specs=pl.BlockSpec((p_blk, n), lambda i: (i, 0)),
        out_shape=jax.ShapeDtypeStruct((m, n), jnp.float32),
    )(x, w)


def _combine_kernel(xx_ref, gmax_ref, y_ref, g_ref, b_ref, out_ref):
    edge = gmax_ref[...] - y_ref[...]
    out_ref[...] = xx_ref[...] + edge * BN_SCALE * g_ref[...] + b_ref[...]


def _combine(xx, gmax, y, g, b, p_blk=1024):
    return pl.pallas_call(
        _combine_kernel,
        grid=(ROWS // p_blk,),
        in_specs=[
            pl.BlockSpec((p_blk, DIM), lambda i: (i, 0)),
            pl.BlockSpec((p_blk, DIM), lambda i: (i, 0)),
            pl.BlockSpec((p_blk, DIM), lambda i: (i, 0)),
            pl.BlockSpec((1, DIM), lambda i: (0, 0)),
            pl.BlockSpec((1, DIM), lambda i: (0, 0)),
        ],
        out_specs=pl.BlockSpec((p_blk, DIM), lambda i: (i, 0)),
        out_shape=jax.ShapeDtypeStruct((ROWS, DIM), jnp.float32),
    )(xx, gmax, y, g, b)


def _res_mlp_kernel(xx_ref, w1_ref, b1_ref, w2_ref, g_ref, b_ref, out_ref):
    xx = xx_ref[...]
    t = _gelu(jnp.dot(xx, w1_ref[...], preferred_element_type=jnp.float32)
              + b1_ref[...])
    t = jnp.dot(t, w2_ref[...], preferred_element_type=jnp.float32)
    out_ref[...] = xx + t * BN_SCALE * g_ref[...] + b_ref[...]


def _res_mlp(xx, w1, b1, w2, g, b, p_blk=1024):
    return pl.pallas_call(
        _res_mlp_kernel,
        grid=(ROWS // p_blk,),
        in_specs=[
            pl.BlockSpec((p_blk, DIM), lambda i: (i, 0)),
            pl.BlockSpec((DIM, 2 * DIM), lambda i: (0, 0)),
            pl.BlockSpec((1, 2 * DIM), lambda i: (0, 0)),
            pl.BlockSpec((2 * DIM, DIM), lambda i: (0, 0)),
            pl.BlockSpec((1, DIM), lambda i: (0, 0)),
            pl.BlockSpec((1, DIM), lambda i: (0, 0)),
        ],
        out_specs=pl.BlockSpec((p_blk, DIM), lambda i: (i, 0)),
        out_shape=jax.ShapeDtypeStruct((ROWS, DIM), jnp.float32),
    )(xx, w1, b1, w2, g, b)


def _final_kernel(xx_ref, g_ref, b_ref, w_ref, out_ref):
    t = xx_ref[...] * BN_SCALE * g_ref[...] + b_ref[...]
    out_ref[...] = jnp.dot(t, w_ref[...], preferred_element_type=jnp.float32)


def _final(xx, g, b, w, p_blk=1024):
    return pl.pallas_call(
        _final_kernel,
        grid=(ROWS // p_blk,),
        in_specs=[
            pl.BlockSpec((p_blk, DIM), lambda i: (i, 0)),
            pl.BlockSpec((1, DIM), lambda i: (0, 0)),
            pl.BlockSpec((1, DIM), lambda i: (0, 0)),
            pl.BlockSpec((DIM, HEAD_DIM), lambda i: (0, 0)),
        ],
        out_specs=pl.BlockSpec((p_blk, HEAD_DIM), lambda i: (i, 0)),
        out_shape=jax.ShapeDtypeStruct((ROWS, HEAD_DIM), jnp.float32),
    )(xx, g, b, w)


def _gather_nbrs(x, idx):
    Bb, Nn, k = idx.shape
    C = x.shape[-1]
    return jnp.take_along_axis(
        x, idx.reshape(Bb, Nn * k, 1), axis=1).reshape(Bb, Nn, k, C)


def kernel(x, xyz, prev_knn, pwd, ne_w1, ne_g1, ne_b1, ne_w2, ne_g2, ne_b2,
           ne_w3, nbr_g, nbr_b, m_w1, m_b1, m_w2, m_g, m_b, lfp_w, lfp_g,
           lfp_b, mm_w1, mm_b1, mm_w2, mm_g, mm_b, pp_g, pp_b, pp_w):
    height = xyz[..., 1:2] / 10.0
    height = height - height.min(axis=1, keepdims=True)
    g8 = jnp.concatenate(
        [xyz, x, height, jnp.zeros_like(height)], axis=-1).reshape(ROWS * 8)
    xyz8 = jnp.pad(xyz.reshape(ROWS, 3), ((0, 0), (0, 5)))

    knng = _sc_topk(pwd.reshape(ROWS, N))
    idx3 = knng.reshape(NW, NG, GR)
    nbr = _sc_gather8(g8, knng.reshape(NW, PPW * K)).reshape(ROWS, K * 8)

    w1p = jnp.pad(ne_w1, ((0, 1), (0, 0)))
    h = _nbr_mlp(nbr, xyz8, w1p, ne_b1[None], ne_w2, ne_b2[None], ne_w3)

    xx = _bn_mlp(h, nbr_g[None], nbr_b[None], m_w1, m_b1[None], m_w2, m_g[None], m_b[None])

    for i in range(4):
        y = _matmul(xx, lfp_w[i])
        gmax = _sc_gather_max(y, idx3)
        xx = _combine(xx, gmax, y, lfp_g[i][None], lfp_b[i][None])
        if i % 2 == 1:
            j = i // 2
            xx = _res_mlp(xx, mm_w1[j], mm_b1[j][None], mm_w2[j], mm_g[j][None], mm_b[j][None])

    return _final(xx, pp_g[None], pp_b[None], pp_w)


# speedup vs baseline: 15.9851x; 1.2212x over previous
import functools
import math

import jax
import jax.numpy as jnp
from jax import lax
from jax.experimental import pallas as pl
from jax.experimental.pallas import tpu as pltpu
from jax.experimental.pallas import tpu_sc as plsc

B, N, K, DIM, HEAD_DIM = 4, 2048, 24, 256, 256
EPS = 1e-5
BN_SCALE = 1.0 / math.sqrt(1.0 + EPS)
ROWS = B * N

NC, NS = 2, 16
NW = NC * NS
PPW = ROWS // NW
PG = 4
NG = PPW // PG
GR = PG * K


def _gmax_body(tab_hbm, idx_hbm, out_hbm, idx_v, rows0, rows1, out_v, sem0,
               sem1):
    wid = lax.axis_index("s") * NC + lax.axis_index("c")
    pltpu.sync_copy(idx_hbm.at[wid], idx_v)
    bufs = (rows0, rows1)
    sems = (sem0, sem1)

    def issue(g, slot):
        pltpu.async_copy(tab_hbm.at[idx_v.at[g]], bufs[slot], sems[slot])

    def wait(slot):
        pltpu.make_async_copy(tab_hbm.at[pl.ds(0, GR)], bufs[slot],
                              sems[slot]).wait()

    def compute(g, slot):
        rows = bufs[slot]

        def p_body(p, _):
            r0 = p * K

            def t_body(t, _):
                c0 = pl.multiple_of(t * 16, 16)
                acc = plsc.bitcast(rows[r0, pl.ds(c0, 16)], jnp.bfloat16)
                for j in range(1, K):
                    acc = jnp.maximum(
                        acc, plsc.bitcast(rows[r0 + j, pl.ds(c0, 16)],
                                          jnp.bfloat16))
                out_v[g * PG + p, pl.ds(c0, 16)] = plsc.bitcast(
                    acc, jnp.int32)
                return 0

            lax.fori_loop(0, DIM // 32, t_body, 0)
            return 0

        lax.fori_loop(0, PG, p_body, 0)

    issue(0, 0)

    def g_body(g2, _):
        g = g2 * 2
        wait(0)
        issue(g + 1, 1)
        compute(g, 0)
        wait(1)

        @pl.when(g + 2 < NG)
        def _():
            issue(g + 2, 0)

        compute(g + 1, 1)
        return 0

    lax.fori_loop(0, NG // 2, g_body, 0)
    pltpu.sync_copy(out_v, out_hbm.at[pl.ds(wid * PPW, PPW)])


def _sc_gather_max(tab, idx3):
    mesh = plsc.VectorSubcoreMesh(core_axis_name="c", subcore_axis_name="s")
    f = pl.kernel(
        _gmax_body,
        mesh=mesh,
        out_type=jax.ShapeDtypeStruct((ROWS, DIM // 2), jnp.int32),
        scratch_types=[
            pltpu.VMEM((NG, GR), jnp.int32),
            pltpu.VMEM((GR, DIM // 2), jnp.int32),
            pltpu.VMEM((GR, DIM // 2), jnp.int32),
            pltpu.VMEM((PPW, DIM // 2), jnp.int32),
            pltpu.SemaphoreType.DMA,
            pltpu.SemaphoreType.DMA,
        ],
        compiler_params=pltpu.CompilerParams(needs_layout_passes=False),
    )
    return f(tab, idx3)


_TKW = 8
_INF = float('inf')


def _merge16_kv(ka, va, kb, vb):
    krb = lax.rev(kb, (0,))
    vrb = lax.rev(vb, (0,))
    cmp = ka <= krb
    lok = jnp.minimum(ka, krb)
    hik = jnp.maximum(ka, krb)
    lov = jnp.where(cmp, va, vrb)
    hiv = jnp.where(cmp, vrb, va)
    lok, lov = plsc.sort_key_val(lok, lov)
    hik, hiv = plsc.sort_key_val(hik, hiv)
    return lok, lov, hik, hiv


def _topk_body(pwd_hbm, out_hbm, buf0, buf1, cval, cidx, outv, sem0, sem1):
    wid = lax.axis_index("s") * NC + lax.axis_index("c")
    row0 = wid * PPW
    boff = (wid // (NW // B)) * N
    ii = jnp.arange(16, dtype=jnp.int32)
    iif = ii.astype(jnp.float32)
    bufs = (buf0, buf1)
    sems = (sem0, sem1)
    nwin = PPW // _TKW

    def issue(g, slot):
        pltpu.async_copy(pwd_hbm.at[pl.ds(row0 + g * _TKW, _TKW)],
                         bufs[slot], sems[slot])

    def wait(slot):
        pltpu.make_async_copy(pwd_hbm.at[pl.ds(0, _TKW)], bufs[slot],
                              sems[slot]).wait()

    def window(g, slot):
        buf = bufs[slot]

        def row_body(r, _):
            def pa(c, carry):
                m1, m2 = carry
                v = buf[r, pl.ds(pl.multiple_of(c * 16, 16), 16)]
                nm1 = jnp.minimum(m1, v)
                m2 = jnp.minimum(m2, jnp.maximum(m1, v))
                return nm1, m2

            m1, m2 = lax.fori_loop(
                0, 128, pa, (jnp.full((16,), _INF), jnp.full((16,), _INF)))
            s1, _ = plsc.sort_key_val(m1, ii)
            s2, _ = plsc.sort_key_val(m2, ii)
            _, _, hik, _ = _merge16_kv(s1, iif, s2, iif)
            tau = jnp.max(jnp.where(ii == 7, hik, -_INF))
            tauv = jnp.full((16,), tau)

            def pb(c, offv):
                cb = pl.multiple_of(c * 16, 16)
                v = buf[r, pl.ds(cb, 16)]
                msk = v <= tauv
                pos = offv + plsc.cumsum(msk.astype(jnp.int32)) - 1
                plsc.store_scatter(cval, [pos], v, mask=msk)
                plsc.store_scatter(cidx, [pos], ii + c * 16, mask=msk)
                return offv + plsc.all_reduce_population_count(msk)

            offv = lax.fori_loop(0, 128, pb, jnp.zeros((16,), jnp.int32))
            cnt = jnp.max(offv)

            nchunk = (cnt + 15) // 16
            rem0 = cnt - (nchunk - 1) * 16

            def sel(ci, carry):
                r0k, r0v, r1k, r1v = carry
                cb = ci * 16
                ck = plsc.load_gather(cval, [cb + ii])
                cv = plsc.load_gather(cidx, [cb + ii])
                nvalid = jnp.where(ci == nchunk - 1, rem0, 16)
                ck = jnp.where(ii < nvalid, ck, _INF)
                ck, cv = plsc.sort_key_val(ck, cv)
                crk = lax.rev(ck, (0,))
                crv = lax.rev(cv, (0,))
                cmp = r1k <= crk
                n1k = jnp.minimum(r1k, crk)
                n1v = jnp.where(cmp, r1v, crv)
                n1k, n1v = plsc.sort_key_val(n1k, n1v)
                return _merge16_kv(r0k, r0v, n1k, n1v)

            init = (jnp.full((16,), _INF), ii, jnp.full((16,), _INF), ii)
            r0k, r0v, r1k, r1v = lax.fori_loop(0, nchunk, sel, init)

            p = g * _TKW + r
            outv[p, pl.ds(0, 16)] = r0v + boff
            plsc.store_scatter(outv.at[p], [ii + 16], r1v + boff, mask=ii < 8)
            return 0

        lax.fori_loop(0, _TKW, row_body, 0)

    issue(0, 0)

    def w_body(g2, _):
        g = g2 * 2
        wait(0)
        issue(g + 1, 1)
        window(g, 0)
        wait(1)

        @pl.when(g + 2 < nwin)
        def _():
            issue(g + 2, 0)

        window(g + 1, 1)
        return 0

    lax.fori_loop(0, nwin // 2, w_body, 0)
    pltpu.sync_copy(outv, out_hbm.at[pl.ds(row0, PPW)])


def _sc_topk(pwd2):
    mesh = plsc.VectorSubcoreMesh(core_axis_name="c", subcore_axis_name="s")
    f = pl.kernel(
        _topk_body,
        mesh=mesh,
        out_type=jax.ShapeDtypeStruct((ROWS, K), jnp.int32),
        scratch_types=[
            pltpu.VMEM((_TKW, N), jnp.float32),
            pltpu.VMEM((_TKW, N), jnp.float32),
            pltpu.VMEM((N,), jnp.float32),
            pltpu.VMEM((N,), jnp.int32),
            pltpu.VMEM((PPW, K), jnp.int32),
            pltpu.SemaphoreType.DMA,
            pltpu.SemaphoreType.DMA,
        ],
        compiler_params=pltpu.CompilerParams(needs_layout_passes=False),
    )
    return f(pwd2)


def _g8_body(tab_hbm, idx_hbm, out_hbm, tab_v, idx_v, out_v, sem):
    wid = lax.axis_index("s") * NC + lax.axis_index("c")
    pltpu.sync_copy(tab_hbm, tab_v)
    pltpu.sync_copy(idx_hbm.at[wid], idx_v)
    ii = jnp.arange(16, dtype=jnp.int32)

    def q_body(q, _):
        a0 = plsc.load_gather(idx_v, [q * 16 + ii]) * 8
        o0 = q * 128 + ii * 8
        for c in range(8):
            g = plsc.load_gather(tab_v, [a0 + c])
            plsc.store_scatter(out_v, [o0 + c], g)
        return 0

    lax.fori_loop(0, (PPW * K) // 16, q_body, 0)
    pltpu.sync_copy(out_v, out_hbm.at[pl.ds(wid * PPW * K * 8, PPW * K * 8)])


def _sc_gather8(tab8, idx2):
    mesh = plsc.VectorSubcoreMesh(core_axis_name="c", subcore_axis_name="s")
    f = pl.kernel(
        _g8_body,
        mesh=mesh,
        out_type=jax.ShapeDtypeStruct((ROWS * K * 8,), jnp.float32),
        scratch_types=[
            pltpu.VMEM((ROWS * 8,), jnp.float32),
            pltpu.VMEM((PPW * K,), jnp.int32),
            pltpu.VMEM((PPW * K * 8,), jnp.float32),
            pltpu.SemaphoreType.DMA,
        ],
        compiler_params=pltpu.CompilerParams(needs_layout_passes=False),
    )
    return f(tab8, idx2)


def _gelu(x):
    return 0.5 * x * (1.0 + lax.erf(x * (1.0 / math.sqrt(2.0))))


def _nbr_mlp_kernel(nbr_ref, c8_ref, w1_ref, b1_ref, w2_ref, b2_ref, w3_ref,
                    out_ref):
    c8 = c8_ref[...]
    hmax = None
    for j in range(K):
        x = nbr_ref[:, j * 8:(j + 1) * 8] - c8
        h = _gelu(jnp.dot(x, w1_ref[...], preferred_element_type=jnp.float32)
                  * BN_SCALE + b1_ref[...])
        h = _gelu(jnp.dot(h, w2_ref[...], preferred_element_type=jnp.float32)
                  * BN_SCALE + b2_ref[...])
        h = jnp.dot(h, w3_ref[...], preferred_element_type=jnp.float32)
        hmax = h if hmax is None else jnp.maximum(hmax, h)
    out_ref[...] = hmax


def _nbr_mlp(nbr, c8, w1, b1, w2, b2, w3, p_blk=512):
    return pl.pallas_call(
        _nbr_mlp_kernel,
        grid=(ROWS // p_blk,),
        in_specs=[
            pl.BlockSpec((p_blk, K * 8), lambda i: (i, 0)),
            pl.BlockSpec((p_blk, 8), lambda i: (i, 0)),
            pl.BlockSpec((8, 16), lambda i: (0, 0)),
            pl.BlockSpec((1, 16), lambda i: (0, 0)),
            pl.BlockSpec((16, 32), lambda i: (0, 0)),
            pl.BlockSpec((1, 32), lambda i: (0, 0)),
            pl.BlockSpec((32, DIM), lambda i: (0, 0)),
        ],
        out_specs=pl.BlockSpec((p_blk, DIM), lambda i: (i, 0)),
        out_shape=jax.ShapeDtypeStruct((ROWS, DIM), jnp.float32),
    )(nbr, c8, w1, b1, w2, b2, w3)


def _bn_mlp_kernel(h_ref, g0_ref, b0_ref, w1_ref, b1_ref, w2_ref, g_ref,
                   b_ref, out_ref):
    xx = h_ref[...] * BN_SCALE * g0_ref[...] + b0_ref[...]
    t = _gelu(jnp.dot(xx, w1_ref[...], preferred_element_type=jnp.float32)
              + b1_ref[...])
    t = jnp.dot(t, w2_ref[...], preferred_element_type=jnp.float32)
    out_ref[...] = xx + t * BN_SCALE * g_ref[...] + b_ref[...]


def _bn_mlp(h, g0, b0, w1, b1, w2, g, b, p_blk=1024):
    return pl.pallas_call(
        _bn_mlp_kernel,
        grid=(ROWS // p_blk,),
        in_specs=[
            pl.BlockSpec((p_blk, DIM), lambda i: (i, 0)),
            pl.BlockSpec((1, DIM), lambda i: (0, 0)),
            pl.BlockSpec((1, DIM), lambda i: (0, 0)),
            pl.BlockSpec((DIM, 2 * DIM), lambda i: (0, 0)),
            pl.BlockSpec((1, 2 * DIM), lambda i: (0, 0)),
            pl.BlockSpec((2 * DIM, DIM), lambda i: (0, 0)),
            pl.BlockSpec((1, DIM), lambda i: (0, 0)),
            pl.BlockSpec((1, DIM), lambda i: (0, 0)),
        ],
        out_specs=pl.BlockSpec((p_blk, DIM), lambda i: (i, 0)),
        out_shape=jax.ShapeDtypeStruct((ROWS, DIM), jnp.float32),
    )(h, g0, b0, w1, b1, w2, g, b)


def _matmul_kernel(x_ref, w_ref, out_ref, outb_ref):
    y = jnp.dot(x_ref[...], w_ref[...], preferred_element_type=jnp.float32)
    out_ref[...] = y
    outb_ref[...] = y.astype(jnp.bfloat16)


def _matmul(x, w, p_blk=1024):
    m, k = x.shape
    _, n = w.shape
    return pl.pallas_call(
        _matmul_kernel,
        grid=(m // p_blk,),
        in_specs=[
            pl.BlockSpec((p_blk, k), lambda i: (i, 0)),
            pl.BlockSpec((k, n), lambda i: (0, 0)),
        ],
        out_specs=[pl.BlockSpec((p_blk, n), lambda i: (i, 0)),
                   pl.BlockSpec((p_blk, n), lambda i: (i, 0))],
        out_shape=[jax.ShapeDtypeStruct((m, n), jnp.float32),
                   jax.ShapeDtypeStruct((m, n), jnp.bfloat16)],
    )(x, w)


def _combine_kernel(xx_ref, gmax_ref, y_ref, g_ref, b_ref, out_ref):
    edge = gmax_ref[...].astype(jnp.float32) - y_ref[...]
    out_ref[...] = xx_ref[...] + edge * BN_SCALE * g_ref[...] + b_ref[...]


def _combine(xx, gmax, y, g, b, p_blk=1024):
    return pl.pallas_call(
        _combine_kernel,
        grid=(ROWS // p_blk,),
        in_specs=[
            pl.BlockSpec((p_blk, DIM), lambda i: (i, 0)),
            pl.BlockSpec((p_blk, DIM), lambda i: (i, 0)),
            pl.BlockSpec((p_blk, DIM), lambda i: (i, 0)),
            pl.BlockSpec((1, DIM), lambda i: (0, 0)),
            pl.BlockSpec((1, DIM), lambda i: (0, 0)),
        ],
        out_specs=pl.BlockSpec((p_blk, DIM), lambda i: (i, 0)),
        out_shape=jax.ShapeDtypeStruct((ROWS, DIM), jnp.float32),
    )(xx, gmax, y, g, b)


def _res_mlp_kernel(xx_ref, w1_ref, b1_ref, w2_ref, g_ref, b_ref, out_ref):
    xx = xx_ref[...]
    t = _gelu(jnp.dot(xx, w1_ref[...], preferred_element_type=jnp.float32)
              + b1_ref[...])
    t = jnp.dot(t, w2_ref[...], preferred_element_type=jnp.float32)
    out_ref[...] = xx + t * BN_SCALE * g_ref[...] + b_ref[...]


def _res_mlp(xx, w1, b1, w2, g, b, p_blk=1024):
    return pl.pallas_call(
        _res_mlp_kernel,
        grid=(ROWS // p_blk,),
        in_specs=[
            pl.BlockSpec((p_blk, DIM), lambda i: (i, 0)),
            pl.BlockSpec((DIM, 2 * DIM), lambda i: (0, 0)),
            pl.BlockSpec((1, 2 * DIM), lambda i: (0, 0)),
            pl.BlockSpec((2 * DIM, DIM), lambda i: (0, 0)),
            pl.BlockSpec((1, DIM), lambda i: (0, 0)),
            pl.BlockSpec((1, DIM), lambda i: (0, 0)),
        ],
        out_specs=pl.BlockSpec((p_blk, DIM), lambda i: (i, 0)),
        out_shape=jax.ShapeDtypeStruct((ROWS, DIM), jnp.float32),
    )(xx, w1, b1, w2, g, b)


def _final_kernel(xx_ref, g_ref, b_ref, w_ref, out_ref):
    t = xx_ref[...] * BN_SCALE * g_ref[...] + b_ref[...]
    out_ref[...] = jnp.dot(t, w_ref[...], preferred_element_type=jnp.float32)


def _final(xx, g, b, w, p_blk=1024):
    return pl.pallas_call(
        _final_kernel,
        grid=(ROWS // p_blk,),
        in_specs=[
            pl.BlockSpec((p_blk, DIM), lambda i: (i, 0)),
            pl.BlockSpec((1, DIM), lambda i: (0, 0)),
            pl.BlockSpec((1, DIM), lambda i: (0, 0)),
            pl.BlockSpec((DIM, HEAD_DIM), lambda i: (0, 0)),
        ],
        out_specs=pl.BlockSpec((p_blk, HEAD_DIM), lambda i: (i, 0)),
        out_shape=jax.ShapeDtypeStruct((ROWS, HEAD_DIM), jnp.float32),
    )(xx, g, b, w)


def _gather_nbrs(x, idx):
    Bb, Nn, k = idx.shape
    C = x.shape[-1]
    return jnp.take_along_axis(
        x, idx.reshape(Bb, Nn * k, 1), axis=1).reshape(Bb, Nn, k, C)


def kernel(x, xyz, prev_knn, pwd, ne_w1, ne_g1, ne_b1, ne_w2, ne_g2, ne_b2,
           ne_w3, nbr_g, nbr_b, m_w1, m_b1, m_w2, m_g, m_b, lfp_w, lfp_g,
           lfp_b, mm_w1, mm_b1, mm_w2, mm_g, mm_b, pp_g, pp_b, pp_w):
    height = xyz[..., 1:2] / 10.0
    height = height - height.min(axis=1, keepdims=True)
    g8 = jnp.concatenate(
        [xyz, x, height, jnp.zeros_like(height)], axis=-1).reshape(ROWS * 8)
    xyz8 = jnp.pad(xyz.reshape(ROWS, 3), ((0, 0), (0, 5)))

    knng = _sc_topk(pwd.reshape(ROWS, N))
    idx3 = knng.reshape(NW, NG, GR)
    nbr = _sc_gather8(g8, knng.reshape(NW, PPW * K)).reshape(ROWS, K * 8)

    w1p = jnp.pad(ne_w1, ((0, 1), (0, 0)))
    h = _nbr_mlp(nbr, xyz8, w1p, ne_b1[None], ne_w2, ne_b2[None], ne_w3)

    xx = _bn_mlp(h, nbr_g[None], nbr_b[None], m_w1, m_b1[None], m_w2, m_g[None], m_b[None])

    for i in range(4):
        y, ybf = _matmul(xx, lfp_w[i])
        ybf32 = lax.bitcast_convert_type(
            ybf.reshape(ROWS, DIM // 2, 2), jnp.int32)
        gmax32 = _sc_gather_max(ybf32, idx3)
        gmax = lax.bitcast_convert_type(
            gmax32, jnp.bfloat16).reshape(ROWS, DIM)
        xx = _combine(xx, gmax, y, lfp_g[i][None], lfp_b[i][None])
        if i % 2 == 1:
            j = i // 2
            xx = _res_mlp(xx, mm_w1[j], mm_b1[j][None], mm_w2[j], mm_g[j][None], mm_b[j][None])

    return _final(xx, pp_g[None], pp_b[None], pp_w)


# speedup vs baseline: 17.3965x; 1.0883x over previous
import functools
import math

import jax
import jax.numpy as jnp
from jax import lax
from jax.experimental import pallas as pl
from jax.experimental.pallas import tpu as pltpu
from jax.experimental.pallas import tpu_sc as plsc

B, N, K, DIM, HEAD_DIM = 4, 2048, 24, 256, 256
EPS = 1e-5
BN_SCALE = 1.0 / math.sqrt(1.0 + EPS)
ROWS = B * N

NC, NS = 2, 16
NW = NC * NS
PPW = ROWS // NW
PG = 4
NG = PPW // PG
GR = PG * K


def _gmax_body(tab_hbm, idx_hbm, out_hbm, idx_v, rows0, rows1, out_v, sem0,
               sem1):
    wid = lax.axis_index("s") * NC + lax.axis_index("c")
    pltpu.sync_copy(idx_hbm.at[wid], idx_v)
    bufs = (rows0, rows1)
    sems = (sem0, sem1)

    def issue(g, slot):
        pltpu.async_copy(tab_hbm.at[idx_v.at[g]], bufs[slot], sems[slot])

    def wait(slot):
        pltpu.make_async_copy(tab_hbm.at[pl.ds(0, GR)], bufs[slot],
                              sems[slot]).wait()

    def compute(g, slot):
        rows = bufs[slot]

        def p_body(p, _):
            r0 = p * K

            def t_body(t, _):
                c0 = pl.multiple_of(t * 16, 16)
                acc = plsc.bitcast(rows[r0, pl.ds(c0, 16)], jnp.bfloat16)
                for j in range(1, K):
                    acc = jnp.maximum(
                        acc, plsc.bitcast(rows[r0 + j, pl.ds(c0, 16)],
                                          jnp.bfloat16))
                out_v[g * PG + p, pl.ds(c0, 16)] = plsc.bitcast(
                    acc, jnp.int32)
                return 0

            lax.fori_loop(0, DIM // 32, t_body, 0)
            return 0

        lax.fori_loop(0, PG, p_body, 0)

    issue(0, 0)

    def g_body(g2, _):
        g = g2 * 2
        wait(0)
        issue(g + 1, 1)
        compute(g, 0)
        wait(1)

        @pl.when(g + 2 < NG)
        def _():
            issue(g + 2, 0)

        compute(g + 1, 1)
        return 0

    lax.fori_loop(0, NG // 2, g_body, 0)
    pltpu.sync_copy(out_v, out_hbm.at[pl.ds(wid * PPW, PPW)])


def _sc_gather_max(tab, idx3):
    mesh = plsc.VectorSubcoreMesh(core_axis_name="c", subcore_axis_name="s")
    f = pl.kernel(
        _gmax_body,
        mesh=mesh,
        out_type=jax.ShapeDtypeStruct((ROWS, DIM // 2), jnp.int32),
        scratch_types=[
            pltpu.VMEM((NG, GR), jnp.int32),
            pltpu.VMEM((GR, DIM // 2), jnp.int32),
            pltpu.VMEM((GR, DIM // 2), jnp.int32),
            pltpu.VMEM((PPW, DIM // 2), jnp.int32),
            pltpu.SemaphoreType.DMA,
            pltpu.SemaphoreType.DMA,
        ],
        compiler_params=pltpu.CompilerParams(needs_layout_passes=False),
    )
    return f(tab, idx3)


_TKW = 8
_INF = float('inf')


def _merge16_kv(ka, va, kb, vb):
    krb = lax.rev(kb, (0,))
    vrb = lax.rev(vb, (0,))
    cmp = ka <= krb
    lok = jnp.minimum(ka, krb)
    hik = jnp.maximum(ka, krb)
    lov = jnp.where(cmp, va, vrb)
    hiv = jnp.where(cmp, vrb, va)
    lok, lov = plsc.sort_key_val(lok, lov)
    hik, hiv = plsc.sort_key_val(hik, hiv)
    return lok, lov, hik, hiv


def _topk_body(pwd_hbm, out_hbm, buf0, buf1, cval, cidx, outv, sem0, sem1):
    wid = lax.axis_index("s") * NC + lax.axis_index("c")
    row0 = wid * PPW
    boff = (wid // (NW // B)) * N
    ii = jnp.arange(16, dtype=jnp.int32)
    iif = ii.astype(jnp.float32)
    bufs = (buf0, buf1)
    sems = (sem0, sem1)
    nwin = PPW // _TKW

    def issue(g, slot):
        pltpu.async_copy(pwd_hbm.at[pl.ds(row0 + g * _TKW, _TKW)],
                         bufs[slot], sems[slot])

    def wait(slot):
        pltpu.make_async_copy(pwd_hbm.at[pl.ds(0, _TKW)], bufs[slot],
                              sems[slot]).wait()

    def window(g, slot):
        buf = bufs[slot]

        def row_body(r, _):
            def pa(c, carry):
                m1, m2 = carry
                for u in range(2):
                    v = buf[r, pl.ds(pl.multiple_of(c * 32 + u * 16, 16), 16)]
                    nm1 = jnp.minimum(m1, v)
                    m2 = jnp.minimum(m2, jnp.maximum(m1, v))
                    m1 = nm1
                return m1, m2

            m1, m2 = lax.fori_loop(
                0, 64, pa, (jnp.full((16,), _INF), jnp.full((16,), _INF)))
            s1, _ = plsc.sort_key_val(m1, ii)
            s2, _ = plsc.sort_key_val(m2, ii)
            _, _, hik, _ = _merge16_kv(s1, iif, s2, iif)
            tau = jnp.max(jnp.where(ii == 7, hik, -_INF))
            tauv = jnp.full((16,), tau)

            def pb(c, offv):
                for u in range(2):
                    cb = pl.multiple_of(c * 32 + u * 16, 16)
                    v = buf[r, pl.ds(cb, 16)]
                    msk = v <= tauv
                    pos = offv + plsc.cumsum(msk.astype(jnp.int32)) - 1
                    plsc.store_scatter(cval, [pos], v, mask=msk)
                    plsc.store_scatter(cidx, [pos], ii + cb, mask=msk)
                    offv = offv + plsc.all_reduce_population_count(msk)
                return offv

            offv = lax.fori_loop(0, 64, pb, jnp.zeros((16,), jnp.int32))
            cnt = jnp.max(offv)

            nchunk = (cnt + 15) // 16
            rem0 = cnt - (nchunk - 1) * 16

            def sel(ci, carry):
                r0k, r0v, r1k, r1v = carry
                cb = ci * 16
                ck = plsc.load_gather(cval, [cb + ii])
                cv = plsc.load_gather(cidx, [cb + ii])
                nvalid = jnp.where(ci == nchunk - 1, rem0, 16)
                ck = jnp.where(ii < nvalid, ck, _INF)
                ck, cv = plsc.sort_key_val(ck, cv)
                crk = lax.rev(ck, (0,))
                crv = lax.rev(cv, (0,))
                cmp = r1k <= crk
                n1k = jnp.minimum(r1k, crk)
                n1v = jnp.where(cmp, r1v, crv)
                n1k, n1v = plsc.sort_key_val(n1k, n1v)
                return _merge16_kv(r0k, r0v, n1k, n1v)

            init = (jnp.full((16,), _INF), ii, jnp.full((16,), _INF), ii)
            r0k, r0v, r1k, r1v = lax.fori_loop(0, nchunk, sel, init)

            p = g * _TKW + r
            outv[p, pl.ds(0, 16)] = r0v + boff
            plsc.store_scatter(outv.at[p], [ii + 16], r1v + boff, mask=ii < 8)
            return 0

        lax.fori_loop(0, _TKW, row_body, 0)

    issue(0, 0)

    def w_body(g2, _):
        g = g2 * 2
        wait(0)
        issue(g + 1, 1)
        window(g, 0)
        wait(1)

        @pl.when(g + 2 < nwin)
        def _():
            issue(g + 2, 0)

        window(g + 1, 1)
        return 0

    lax.fori_loop(0, nwin // 2, w_body, 0)
    pltpu.sync_copy(outv, out_hbm.at[pl.ds(row0, PPW)])


def _sc_topk(pwd2):
    mesh = plsc.VectorSubcoreMesh(core_axis_name="c", subcore_axis_name="s")
    f = pl.kernel(
        _topk_body,
        mesh=mesh,
        out_type=jax.ShapeDtypeStruct((ROWS, K), jnp.int32),
        scratch_types=[
            pltpu.VMEM((_TKW, N), jnp.float32),
            pltpu.VMEM((_TKW, N), jnp.float32),
            pltpu.VMEM((N,), jnp.float32),
            pltpu.VMEM((N,), jnp.int32),
            pltpu.VMEM((PPW, K), jnp.int32),
            pltpu.SemaphoreType.DMA,
            pltpu.SemaphoreType.DMA,
        ],
        compiler_params=pltpu.CompilerParams(needs_layout_passes=False),
    )
    return f(pwd2)


def _g8_body(tab_hbm, idx_hbm, out_hbm, tab_v, idx_v, out_v, sem):
    wid = lax.axis_index("s") * NC + lax.axis_index("c")
    pltpu.sync_copy(tab_hbm, tab_v)
    pltpu.sync_copy(idx_hbm.at[wid], idx_v)
    ii = jnp.arange(16, dtype=jnp.int32)

    def q_body(q, _):
        a0 = plsc.load_gather(idx_v, [q * 16 + ii]) * 8
        o0 = q * 128 + ii * 8
        for c in range(8):
            g = plsc.load_gather(tab_v, [a0 + c])
            plsc.store_scatter(out_v, [o0 + c], g)
        return 0

    lax.fori_loop(0, (PPW * K) // 16, q_body, 0)
    pltpu.sync_copy(out_v, out_hbm.at[pl.ds(wid * PPW * K * 8, PPW * K * 8)])


def _sc_gather8(tab8, idx2):
    mesh = plsc.VectorSubcoreMesh(core_axis_name="c", subcore_axis_name="s")
    f = pl.kernel(
        _g8_body,
        mesh=mesh,
        out_type=jax.ShapeDtypeStruct((ROWS * K * 8,), jnp.float32),
        scratch_types=[
            pltpu.VMEM((ROWS * 8,), jnp.float32),
            pltpu.VMEM((PPW * K,), jnp.int32),
            pltpu.VMEM((PPW * K * 8,), jnp.float32),
            pltpu.SemaphoreType.DMA,
        ],
        compiler_params=pltpu.CompilerParams(needs_layout_passes=False),
    )
    return f(tab8, idx2)


def _gelu(x):
    return 0.5 * x * (1.0 + lax.erf(x * (1.0 / math.sqrt(2.0))))


def _nbr_mlp_kernel(nbr_ref, c8_ref, w1_ref, b1_ref, w2_ref, b2_ref, w3_ref,
                    out_ref):
    c8 = c8_ref[...]
    hmax = None
    for j in range(K):
        x = nbr_ref[:, j * 8:(j + 1) * 8] - c8
        h = _gelu(jnp.dot(x, w1_ref[...], preferred_element_type=jnp.float32)
                  * BN_SCALE + b1_ref[...])
        h = _gelu(jnp.dot(h, w2_ref[...], preferred_element_type=jnp.float32)
                  * BN_SCALE + b2_ref[...])
        h = jnp.dot(h, w3_ref[...], preferred_element_type=jnp.float32)
        hmax = h if hmax is None else jnp.maximum(hmax, h)
    out_ref[...] = hmax


def _nbr_mlp(nbr, c8, w1, b1, w2, b2, w3, p_blk=512):
    return pl.pallas_call(
        _nbr_mlp_kernel,
        grid=(ROWS // p_blk,),
        in_specs=[
            pl.BlockSpec((p_blk, K * 8), lambda i: (i, 0)),
            pl.BlockSpec((p_blk, 8), lambda i: (i, 0)),
            pl.BlockSpec((8, 16), lambda i: (0, 0)),
            pl.BlockSpec((1, 16), lambda i: (0, 0)),
            pl.BlockSpec((16, 32), lambda i: (0, 0)),
            pl.BlockSpec((1, 32), lambda i: (0, 0)),
            pl.BlockSpec((32, DIM), lambda i: (0, 0)),
        ],
        out_specs=pl.BlockSpec((p_blk, DIM), lambda i: (i, 0)),
        out_shape=jax.ShapeDtypeStruct((ROWS, DIM), jnp.float32),
    )(nbr, c8, w1, b1, w2, b2, w3)


_ROW_SPEC = pl.BlockSpec((1024, DIM), lambda i: (i, 0))
_VEC_SPEC = pl.BlockSpec((1, DIM), lambda i: (0, 0))
_VEC2_SPEC = pl.BlockSpec((1, 2 * DIM), lambda i: (0, 0))
_W_SPEC = pl.BlockSpec((DIM, DIM), lambda i: (0, 0))
_W12_SPEC = pl.BlockSpec((DIM, 2 * DIM), lambda i: (0, 0))
_W21_SPEC = pl.BlockSpec((2 * DIM, DIM), lambda i: (0, 0))
_YOUT = [jax.ShapeDtypeStruct((ROWS, DIM), jnp.float32),
         jax.ShapeDtypeStruct((ROWS, DIM), jnp.float32),
         jax.ShapeDtypeStruct((ROWS, DIM), jnp.bfloat16)]
_YOUT_SPEC = [_ROW_SPEC, _ROW_SPEC, _ROW_SPEC]


def _mlp_block(xx, w1, b1, w2, g, b):
    t = _gelu(jnp.dot(xx, w1, preferred_element_type=jnp.float32) + b1)
    t = jnp.dot(t, w2, preferred_element_type=jnp.float32)
    return xx + t * BN_SCALE * g + b


def _emit_y(xx, w_n, xx_ref, y_ref, yb_ref):
    y = jnp.dot(xx, w_n, preferred_element_type=jnp.float32)
    xx_ref[...] = xx
    y_ref[...] = y
    yb_ref[...] = y.astype(jnp.bfloat16)


def _bn_mlp_y_kernel(h_ref, g0_ref, b0_ref, w1_ref, b1_ref, w2_ref, g_ref,
                     b_ref, wn_ref, xx_ref, y_ref, yb_ref):
    xx = h_ref[...] * BN_SCALE * g0_ref[...] + b0_ref[...]
    xx = _mlp_block(xx, w1_ref[...], b1_ref[...], w2_ref[...], g_ref[...],
                    b_ref[...])
    _emit_y(xx, wn_ref[...], xx_ref, y_ref, yb_ref)


def _bn_mlp_y(h, g0, b0, w1, b1, w2, g, b, wn):
    return pl.pallas_call(
        _bn_mlp_y_kernel,
        grid=(8,),
        in_specs=[_ROW_SPEC, _VEC_SPEC, _VEC_SPEC, _W12_SPEC, _VEC2_SPEC,
                  _W21_SPEC, _VEC_SPEC, _VEC_SPEC, _W_SPEC],
        out_specs=_YOUT_SPEC,
        out_shape=_YOUT,
    )(h, g0, b0, w1, b1, w2, g, b, wn)


def _combine(xx_ref, gmax_ref, y_ref, g_ref, b_ref):
    edge = gmax_ref[...].astype(jnp.float32) - y_ref[...]
    return xx_ref[...] + edge * BN_SCALE * g_ref[...] + b_ref[...]


def _comb_y_kernel(xx_ref, gmax_ref, y_ref, g_ref, b_ref, wn_ref, xxo_ref,
                   yo_ref, ybo_ref):
    xx = _combine(xx_ref, gmax_ref, y_ref, g_ref, b_ref)
    _emit_y(xx, wn_ref[...], xxo_ref, yo_ref, ybo_ref)


def _comb_y(xx, gmax, y, g, b, wn):
    return pl.pallas_call(
        _comb_y_kernel,
        grid=(8,),
        in_specs=[_ROW_SPEC, _ROW_SPEC, _ROW_SPEC, _VEC_SPEC, _VEC_SPEC,
                  _W_SPEC],
        out_specs=_YOUT_SPEC,
        out_shape=_YOUT,
    )(xx, gmax, y, g, b, wn)


def _comb_mm_y_kernel(xx_ref, gmax_ref, y_ref, g_ref, b_ref, w1_ref, b1_ref,
                      w2_ref, mg_ref, mb_ref, wn_ref, xxo_ref, yo_ref,
                      ybo_ref):
    xx = _combine(xx_ref, gmax_ref, y_ref, g_ref, b_ref)
    xx = _mlp_block(xx, w1_ref[...], b1_ref[...], w2_ref[...], mg_ref[...],
                    mb_ref[...])
    _emit_y(xx, wn_ref[...], xxo_ref, yo_ref, ybo_ref)


def _comb_mm_y(xx, gmax, y, g, b, w1, b1, w2, mg, mb, wn):
    return pl.pallas_call(
        _comb_mm_y_kernel,
        grid=(8,),
        in_specs=[_ROW_SPEC, _ROW_SPEC, _ROW_SPEC, _VEC_SPEC, _VEC_SPEC,
                  _W12_SPEC, _VEC2_SPEC, _W21_SPEC, _VEC_SPEC, _VEC_SPEC,
                  _W_SPEC],
        out_specs=_YOUT_SPEC,
        out_shape=_YOUT,
    )(xx, gmax, y, g, b, w1, b1, w2, mg, mb, wn)


def _comb_mm_fin_kernel(xx_ref, gmax_ref, y_ref, g_ref, b_ref, w1_ref,
                        b1_ref, w2_ref, mg_ref, mb_ref, pg_ref, pb_ref,
                        pw_ref, out_ref):
    xx = _combine(xx_ref, gmax_ref, y_ref, g_ref, b_ref)
    xx = _mlp_block(xx, w1_ref[...], b1_ref[...], w2_ref[...], mg_ref[...],
                    mb_ref[...])
    t = xx * BN_SCALE * pg_ref[...] + pb_ref[...]
    out_ref[...] = jnp.dot(t, pw_ref[...], preferred_element_type=jnp.float32)


def _comb_mm_fin(xx, gmax, y, g, b, w1, b1, w2, mg, mb, pg, pb, pw):
    return pl.pallas_call(
        _comb_mm_fin_kernel,
        grid=(8,),
        in_specs=[_ROW_SPEC, _ROW_SPEC, _ROW_SPEC, _VEC_SPEC, _VEC_SPEC,
                  _W12_SPEC, _VEC2_SPEC, _W21_SPEC, _VEC_SPEC, _VEC_SPEC,
                  _VEC_SPEC, _VEC_SPEC,
                  pl.BlockSpec((DIM, HEAD_DIM), lambda i: (0, 0))],
        out_specs=pl.BlockSpec((1024, HEAD_DIM), lambda i: (i, 0)),
        out_shape=jax.ShapeDtypeStruct((ROWS, HEAD_DIM), jnp.float32),
    )(xx, gmax, y, g, b, w1, b1, w2, mg, mb, pg, pb, pw)


def _gather_nbrs(x, idx):
    Bb, Nn, k = idx.shape
    C = x.shape[-1]
    return jnp.take_along_axis(
        x, idx.reshape(Bb, Nn * k, 1), axis=1).reshape(Bb, Nn, k, C)


def kernel(x, xyz, prev_knn, pwd, ne_w1, ne_g1, ne_b1, ne_w2, ne_g2, ne_b2,
           ne_w3, nbr_g, nbr_b, m_w1, m_b1, m_w2, m_g, m_b, lfp_w, lfp_g,
           lfp_b, mm_w1, mm_b1, mm_w2, mm_g, mm_b, pp_g, pp_b, pp_w):
    height = xyz[..., 1:2] / 10.0
    height = height - height.min(axis=1, keepdims=True)
    g8 = jnp.concatenate(
        [xyz, x, height, jnp.zeros_like(height)], axis=-1).reshape(ROWS * 8)
    xyz8 = jnp.pad(xyz.reshape(ROWS, 3), ((0, 0), (0, 5)))

    knng = _sc_topk(pwd.reshape(ROWS, N))
    idx3 = knng.reshape(NW, NG, GR)
    nbr = _sc_gather8(g8, knng.reshape(NW, PPW * K)).reshape(ROWS, K * 8)

    w1p = jnp.pad(ne_w1, ((0, 1), (0, 0)))
    h = _nbr_mlp(nbr, xyz8, w1p, ne_b1[None], ne_w2, ne_b2[None], ne_w3)

    xx, y, ybf = _bn_mlp_y(h, nbr_g[None], nbr_b[None], m_w1, m_b1[None],
                           m_w2, m_g[None], m_b[None], lfp_w[0])

    def gmax_of(ybf):
        ybf32 = lax.bitcast_convert_type(
            ybf.reshape(ROWS, DIM // 2, 2), jnp.int32)
        return lax.bitcast_convert_type(
            _sc_gather_max(ybf32, idx3), jnp.bfloat16).reshape(ROWS, DIM)

    gmax = gmax_of(ybf)
    xx, y, ybf = _comb_y(xx, gmax, y, lfp_g[0][None], lfp_b[0][None],
                         lfp_w[1])
    gmax = gmax_of(ybf)
    xx, y, ybf = _comb_mm_y(xx, gmax, y, lfp_g[1][None], lfp_b[1][None],
                            mm_w1[0], mm_b1[0][None], mm_w2[0], mm_g[0][None],
                            mm_b[0][None], lfp_w[2])
    gmax = gmax_of(ybf)
    xx, y, ybf = _comb_y(xx, gmax, y, lfp_g[2][None], lfp_b[2][None],
                         lfp_w[3])
    gmax = gmax_of(ybf)
    return _comb_mm_fin(xx, gmax, y, lfp_g[3][None], lfp_b[3][None],
                        mm_w1[1], mm_b1[1][None], mm_w2[1], mm_g[1][None],
                        mm_b[1][None], pp_g[None], pp_b[None], pp_w)


# speedup vs baseline: 17.5147x; 1.0068x over previous
import functools
import math

import jax
import jax.numpy as jnp
from jax import lax
from jax.experimental import pallas as pl
from jax.experimental.pallas import tpu as pltpu
from jax.experimental.pallas import tpu_sc as plsc

B, N, K, DIM, HEAD_DIM = 4, 2048, 24, 256, 256
EPS = 1e-5
BN_SCALE = 1.0 / math.sqrt(1.0 + EPS)
ROWS = B * N

NC, NS = 2, 16
NW = NC * NS
PPW = ROWS // NW
PG = 4
NG = PPW // PG
GR = PG * K


def _gmax_body(tab_hbm, idx_hbm, out_hbm, idx_v, rows0, rows1, out_v, sem0,
               sem1):
    wid = lax.axis_index("s") * NC + lax.axis_index("c")
    pltpu.sync_copy(idx_hbm.at[wid], idx_v)
    bufs = (rows0, rows1)
    sems = (sem0, sem1)

    def issue(g, slot):
        pltpu.async_copy(tab_hbm.at[idx_v.at[g]], bufs[slot], sems[slot])

    def wait(slot):
        pltpu.make_async_copy(tab_hbm.at[pl.ds(0, GR)], bufs[slot],
                              sems[slot]).wait()

    def compute(g, slot):
        rows = bufs[slot]

        def p_body(p, _):
            r0 = p * K

            def t_body(t, _):
                c0 = pl.multiple_of(t * 16, 16)
                acc = plsc.bitcast(rows[r0, pl.ds(c0, 16)], jnp.bfloat16)
                for j in range(1, K):
                    acc = jnp.maximum(
                        acc, plsc.bitcast(rows[r0 + j, pl.ds(c0, 16)],
                                          jnp.bfloat16))
                out_v[g * PG + p, pl.ds(c0, 16)] = plsc.bitcast(
                    acc, jnp.int32)
                return 0

            lax.fori_loop(0, DIM // 32, t_body, 0)
            return 0

        lax.fori_loop(0, PG, p_body, 0)

    issue(0, 0)

    def g_body(g2, _):
        g = g2 * 2
        wait(0)
        issue(g + 1, 1)
        compute(g, 0)
        wait(1)

        @pl.when(g + 2 < NG)
        def _():
            issue(g + 2, 0)

        compute(g + 1, 1)
        return 0

    lax.fori_loop(0, NG // 2, g_body, 0)
    pltpu.sync_copy(out_v, out_hbm.at[pl.ds(wid * PPW, PPW)])


def _sc_gather_max(tab, idx3):
    mesh = plsc.VectorSubcoreMesh(core_axis_name="c", subcore_axis_name="s")
    f = pl.kernel(
        _gmax_body,
        mesh=mesh,
        out_type=jax.ShapeDtypeStruct((ROWS, DIM // 2), jnp.int32),
        scratch_types=[
            pltpu.VMEM((NG, GR), jnp.int32),
            pltpu.VMEM((GR, DIM // 2), jnp.int32),
            pltpu.VMEM((GR, DIM // 2), jnp.int32),
            pltpu.VMEM((PPW, DIM // 2), jnp.int32),
            pltpu.SemaphoreType.DMA,
            pltpu.SemaphoreType.DMA,
        ],
        compiler_params=pltpu.CompilerParams(needs_layout_passes=False),
    )
    return f(tab, idx3)


_TKW = 4
_INF = float('inf')


def _merge16_kv(ka, va, kb, vb):
    krb = lax.rev(kb, (0,))
    vrb = lax.rev(vb, (0,))
    cmp = ka <= krb
    lok = jnp.minimum(ka, krb)
    hik = jnp.maximum(ka, krb)
    lov = jnp.where(cmp, va, vrb)
    hiv = jnp.where(cmp, vrb, va)
    lok, lov = plsc.sort_key_val(lok, lov)
    hik, hiv = plsc.sort_key_val(hik, hiv)
    return lok, lov, hik, hiv


def _topk_body(pwd_hbm, g8_hbm, out_hbm, nbr_hbm, buf0, buf1, cval, cidx,
               outv, tab_v, nbrw, sem0, sem1):
    wid = lax.axis_index("s") * NC + lax.axis_index("c")
    row0 = wid * PPW
    boff = (wid // (NW // B)) * N
    ii = jnp.arange(16, dtype=jnp.int32)
    iif = ii.astype(jnp.float32)
    bufs = (buf0, buf1)
    sems = (sem0, sem1)
    nwin = PPW // _TKW
    pltpu.sync_copy(g8_hbm, tab_v)

    def issue(g, slot):
        pltpu.async_copy(pwd_hbm.at[pl.ds(row0 + g * _TKW, _TKW)],
                         bufs[slot], sems[slot])

    def wait(slot):
        pltpu.make_async_copy(pwd_hbm.at[pl.ds(0, _TKW)], bufs[slot],
                              sems[slot]).wait()

    def window(g, slot):
        buf = bufs[slot]

        def row_body(r, _):
            def pa(c, carry):
                m1, m2 = carry
                for u in range(2):
                    v = buf[r, pl.ds(pl.multiple_of(c * 32 + u * 16, 16), 16)]
                    nm1 = jnp.minimum(m1, v)
                    m2 = jnp.minimum(m2, jnp.maximum(m1, v))
                    m1 = nm1
                return m1, m2

            m1, m2 = lax.fori_loop(
                0, 64, pa, (jnp.full((16,), _INF), jnp.full((16,), _INF)))
            s1, _ = plsc.sort_key_val(m1, ii)
            s2, _ = plsc.sort_key_val(m2, ii)
            _, _, hik, _ = _merge16_kv(s1, iif, s2, iif)
            tau = jnp.max(jnp.where(ii == 7, hik, -_INF))
            tauv = jnp.full((16,), tau)

            def pb(c, offv):
                for u in range(2):
                    cb = pl.multiple_of(c * 32 + u * 16, 16)
                    v = buf[r, pl.ds(cb, 16)]
                    msk = v <= tauv
                    pos = offv + plsc.cumsum(msk.astype(jnp.int32)) - 1
                    plsc.store_scatter(cval, [pos], v, mask=msk)
                    plsc.store_scatter(cidx, [pos], ii + cb, mask=msk)
                    offv = offv + plsc.all_reduce_population_count(msk)
                return offv

            offv = lax.fori_loop(0, 64, pb, jnp.zeros((16,), jnp.int32))
            cnt = jnp.max(offv)

            nchunk = (cnt + 15) // 16
            rem0 = cnt - (nchunk - 1) * 16

            def sel(ci, carry):
                r0k, r0v, r1k, r1v = carry
                cb = ci * 16
                ck = plsc.load_gather(cval, [cb + ii])
                cv = plsc.load_gather(cidx, [cb + ii])
                nvalid = jnp.where(ci == nchunk - 1, rem0, 16)
                ck = jnp.where(ii < nvalid, ck, _INF)
                ck, cv = plsc.sort_key_val(ck, cv)
                crk = lax.rev(ck, (0,))
                crv = lax.rev(cv, (0,))
                cmp = r1k <= crk
                n1k = jnp.minimum(r1k, crk)
                n1v = jnp.where(cmp, r1v, crv)
                n1k, n1v = plsc.sort_key_val(n1k, n1v)
                return _merge16_kv(r0k, r0v, n1k, n1v)

            init = (jnp.full((16,), _INF), ii, jnp.full((16,), _INF), ii)
            r0k, r0v, r1k, r1v = lax.fori_loop(0, nchunk, sel, init)

            p = g * _TKW + r
            gid0 = r0v + boff
            gid1 = r1v + boff
            outv[p, pl.ds(0, 16)] = gid0
            plsc.store_scatter(outv.at[p], [ii + 16], gid1, mask=ii < 8)

            a1 = gid0 * 8
            a2 = gid1 * 8
            o1 = r * (K * 8) + ii * 8
            o2 = o1 + 128
            m8 = ii < 8
            for c in range(8):
                g1 = plsc.load_gather(tab_v, [a1 + c])
                g2 = plsc.load_gather(tab_v, [a2 + c], mask=m8)
                plsc.store_scatter(nbrw, [o1 + c], g1)
                plsc.store_scatter(nbrw, [o2 + c], g2, mask=m8)
            return 0

        lax.fori_loop(0, _TKW, row_body, 0)
        pltpu.sync_copy(
            nbrw, nbr_hbm.at[pl.ds((row0 + g * _TKW) * (K * 8),
                                   _TKW * K * 8)])

    issue(0, 0)

    def w_body(g2, _):
        g = g2 * 2
        wait(0)
        issue(g + 1, 1)
        window(g, 0)
        wait(1)

        @pl.when(g + 2 < nwin)
        def _():
            issue(g + 2, 0)

        window(g + 1, 1)
        return 0

    lax.fori_loop(0, nwin // 2, w_body, 0)
    pltpu.sync_copy(outv, out_hbm.at[pl.ds(row0, PPW)])


def _sc_topk(pwd2, g8f):
    mesh = plsc.VectorSubcoreMesh(core_axis_name="c", subcore_axis_name="s")
    f = pl.kernel(
        _topk_body,
        mesh=mesh,
        out_type=(jax.ShapeDtypeStruct((ROWS, K), jnp.int32),
                  jax.ShapeDtypeStruct((ROWS * K * 8,), jnp.float32)),
        scratch_types=[
            pltpu.VMEM((_TKW, N), jnp.float32),
            pltpu.VMEM((_TKW, N), jnp.float32),
            pltpu.VMEM((N,), jnp.float32),
            pltpu.VMEM((N,), jnp.int32),
            pltpu.VMEM((PPW, K), jnp.int32),
            pltpu.VMEM((ROWS * 8,), jnp.float32),
            pltpu.VMEM((_TKW * K * 8,), jnp.float32),
            pltpu.SemaphoreType.DMA,
            pltpu.SemaphoreType.DMA,
        ],
        compiler_params=pltpu.CompilerParams(needs_layout_passes=False),
    )
    return f(pwd2, g8f)


def _g8_body(tab_hbm, idx_hbm, out_hbm, tab_v, idx_v, out_v, sem):
    wid = lax.axis_index("s") * NC + lax.axis_index("c")
    pltpu.sync_copy(tab_hbm, tab_v)
    pltpu.sync_copy(idx_hbm.at[wid], idx_v)
    ii = jnp.arange(16, dtype=jnp.int32)

    def q_body(q, _):
        a0 = plsc.load_gather(idx_v, [q * 16 + ii]) * 8
        o0 = q * 128 + ii * 8
        for c in range(8):
            g = plsc.load_gather(tab_v, [a0 + c])
            plsc.store_scatter(out_v, [o0 + c], g)
        return 0

    lax.fori_loop(0, (PPW * K) // 16, q_body, 0)
    pltpu.sync_copy(out_v, out_hbm.at[pl.ds(wid * PPW * K * 8, PPW * K * 8)])


def _sc_gather8(tab8, idx2):
    mesh = plsc.VectorSubcoreMesh(core_axis_name="c", subcore_axis_name="s")
    f = pl.kernel(
        _g8_body,
        mesh=mesh,
        out_type=jax.ShapeDtypeStruct((ROWS * K * 8,), jnp.float32),
        scratch_types=[
            pltpu.VMEM((ROWS * 8,), jnp.float32),
            pltpu.VMEM((PPW * K,), jnp.int32),
            pltpu.VMEM((PPW * K * 8,), jnp.float32),
            pltpu.SemaphoreType.DMA,
        ],
        compiler_params=pltpu.CompilerParams(needs_layout_passes=False),
    )
    return f(tab8, idx2)


def _gelu(x):
    return 0.5 * x * (1.0 + lax.erf(x * (1.0 / math.sqrt(2.0))))


def _nbr_mlp_kernel(nbr_ref, c8_ref, w1_ref, b1_ref, w2_ref, b2_ref, w3_ref,
                    out_ref):
    c8 = c8_ref[...]
    hmax = None
    for j in range(K):
        x = nbr_ref[:, j * 8:(j + 1) * 8] - c8
        h = _gelu(jnp.dot(x, w1_ref[...], preferred_element_type=jnp.float32)
                  * BN_SCALE + b1_ref[...])
        h = _gelu(jnp.dot(h, w2_ref[...], preferred_element_type=jnp.float32)
                  * BN_SCALE + b2_ref[...])
        h = jnp.dot(h, w3_ref[...], preferred_element_type=jnp.float32)
        hmax = h if hmax is None else jnp.maximum(hmax, h)
    out_ref[...] = hmax


def _nbr_mlp(nbr, c8, w1, b1, w2, b2, w3, p_blk=512):
    return pl.pallas_call(
        _nbr_mlp_kernel,
        grid=(ROWS // p_blk,),
        in_specs=[
            pl.BlockSpec((p_blk, K * 8), lambda i: (i, 0)),
            pl.BlockSpec((p_blk, 8), lambda i: (i, 0)),
            pl.BlockSpec((8, 16), lambda i: (0, 0)),
            pl.BlockSpec((1, 16), lambda i: (0, 0)),
            pl.BlockSpec((16, 32), lambda i: (0, 0)),
            pl.BlockSpec((1, 32), lambda i: (0, 0)),
            pl.BlockSpec((32, DIM), lambda i: (0, 0)),
        ],
        out_specs=pl.BlockSpec((p_blk, DIM), lambda i: (i, 0)),
        out_shape=jax.ShapeDtypeStruct((ROWS, DIM), jnp.float32),
    )(nbr, c8, w1, b1, w2, b2, w3)


_ROW_SPEC = pl.BlockSpec((1024, DIM), lambda i: (i, 0))
_VEC_SPEC = pl.BlockSpec((1, DIM), lambda i: (0, 0))
_VEC2_SPEC = pl.BlockSpec((1, 2 * DIM), lambda i: (0, 0))
_W_SPEC = pl.BlockSpec((DIM, DIM), lambda i: (0, 0))
_W12_SPEC = pl.BlockSpec((DIM, 2 * DIM), lambda i: (0, 0))
_W21_SPEC = pl.BlockSpec((2 * DIM, DIM), lambda i: (0, 0))
_YOUT = [jax.ShapeDtypeStruct((ROWS, DIM), jnp.float32),
         jax.ShapeDtypeStruct((ROWS, DIM), jnp.float32),
         jax.ShapeDtypeStruct((ROWS, DIM), jnp.bfloat16)]
_YOUT_SPEC = [_ROW_SPEC, _ROW_SPEC, _ROW_SPEC]


def _mlp_block(xx, w1, b1, w2, g, b):
    t = _gelu(jnp.dot(xx, w1, preferred_element_type=jnp.float32) + b1)
    t = jnp.dot(t, w2, preferred_element_type=jnp.float32)
    return xx + t * BN_SCALE * g + b


def _emit_y(xx, w_n, xx_ref, y_ref, yb_ref):
    y = jnp.dot(xx, w_n, preferred_element_type=jnp.float32)
    xx_ref[...] = xx
    y_ref[...] = y
    yb_ref[...] = y.astype(jnp.bfloat16)


def _bn_mlp_y_kernel(h_ref, g0_ref, b0_ref, w1_ref, b1_ref, w2_ref, g_ref,
                     b_ref, wn_ref, xx_ref, y_ref, yb_ref):
    xx = h_ref[...] * BN_SCALE * g0_ref[...] + b0_ref[...]
    xx = _mlp_block(xx, w1_ref[...], b1_ref[...], w2_ref[...], g_ref[...],
                    b_ref[...])
    _emit_y(xx, wn_ref[...], xx_ref, y_ref, yb_ref)


def _bn_mlp_y(h, g0, b0, w1, b1, w2, g, b, wn):
    return pl.pallas_call(
        _bn_mlp_y_kernel,
        grid=(8,),
        in_specs=[_ROW_SPEC, _VEC_SPEC, _VEC_SPEC, _W12_SPEC, _VEC2_SPEC,
                  _W21_SPEC, _VEC_SPEC, _VEC_SPEC, _W_SPEC],
        out_specs=_YOUT_SPEC,
        out_shape=_YOUT,
    )(h, g0, b0, w1, b1, w2, g, b, wn)


def _combine(xx_ref, gmax_ref, y_ref, g_ref, b_ref):
    edge = gmax_ref[...].astype(jnp.float32) - y_ref[...]
    return xx_ref[...] + edge * BN_SCALE * g_ref[...] + b_ref[...]


def _comb_y_kernel(xx_ref, gmax_ref, y_ref, g_ref, b_ref, wn_ref, xxo_ref,
                   yo_ref, ybo_ref):
    xx = _combine(xx_ref, gmax_ref, y_ref, g_ref, b_ref)
    _emit_y(xx, wn_ref[...], xxo_ref, yo_ref, ybo_ref)


def _comb_y(xx, gmax, y, g, b, wn):
    return pl.pallas_call(
        _comb_y_kernel,
        grid=(8,),
        in_specs=[_ROW_SPEC, _ROW_SPEC, _ROW_SPEC, _VEC_SPEC, _VEC_SPEC,
                  _W_SPEC],
        out_specs=_YOUT_SPEC,
        out_shape=_YOUT,
    )(xx, gmax, y, g, b, wn)


def _comb_mm_y_kernel(xx_ref, gmax_ref, y_ref, g_ref, b_ref, w1_ref, b1_ref,
                      w2_ref, mg_ref, mb_ref, wn_ref, xxo_ref, yo_ref,
                      ybo_ref):
    xx = _combine(xx_ref, gmax_ref, y_ref, g_ref, b_ref)
    xx = _mlp_block(xx, w1_ref[...], b1_ref[...], w2_ref[...], mg_ref[...],
                    mb_ref[...])
    _emit_y(xx, wn_ref[...], xxo_ref, yo_ref, ybo_ref)


def _comb_mm_y(xx, gmax, y, g, b, w1, b1, w2, mg, mb, wn):
    return pl.pallas_call(
        _comb_mm_y_kernel,
        grid=(8,),
        in_specs=[_ROW_SPEC, _ROW_SPEC, _ROW_SPEC, _VEC_SPEC, _VEC_SPEC,
                  _W12_SPEC, _VEC2_SPEC, _W21_SPEC, _VEC_SPEC, _VEC_SPEC,
                  _W_SPEC],
        out_specs=_YOUT_SPEC,
        out_shape=_YOUT,
    )(xx, gmax, y, g, b, w1, b1, w2, mg, mb, wn)


def _comb_mm_fin_kernel(xx_ref, gmax_ref, y_ref, g_ref, b_ref, w1_ref,
                        b1_ref, w2_ref, mg_ref, mb_ref, pg_ref, pb_ref,
                        pw_ref, out_ref):
    xx = _combine(xx_ref, gmax_ref, y_ref, g_ref, b_ref)
    xx = _mlp_block(xx, w1_ref[...], b1_ref[...], w2_ref[...], mg_ref[...],
                    mb_ref[...])
    t = xx * BN_SCALE * pg_ref[...] + pb_ref[...]
    out_ref[...] = jnp.dot(t, pw_ref[...], preferred_element_type=jnp.float32)


def _comb_mm_fin(xx, gmax, y, g, b, w1, b1, w2, mg, mb, pg, pb, pw):
    return pl.pallas_call(
        _comb_mm_fin_kernel,
        grid=(8,),
        in_specs=[_ROW_SPEC, _ROW_SPEC, _ROW_SPEC, _VEC_SPEC, _VEC_SPEC,
                  _W12_SPEC, _VEC2_SPEC, _W21_SPEC, _VEC_SPEC, _VEC_SPEC,
                  _VEC_SPEC, _VEC_SPEC,
                  pl.BlockSpec((DIM, HEAD_DIM), lambda i: (0, 0))],
        out_specs=pl.BlockSpec((1024, HEAD_DIM), lambda i: (i, 0)),
        out_shape=jax.ShapeDtypeStruct((ROWS, HEAD_DIM), jnp.float32),
    )(xx, gmax, y, g, b, w1, b1, w2, mg, mb, pg, pb, pw)


def _gather_nbrs(x, idx):
    Bb, Nn, k = idx.shape
    C = x.shape[-1]
    return jnp.take_along_axis(
        x, idx.reshape(Bb, Nn * k, 1), axis=1).reshape(Bb, Nn, k, C)


def kernel(x, xyz, prev_knn, pwd, ne_w1, ne_g1, ne_b1, ne_w2, ne_g2, ne_b2,
           ne_w3, nbr_g, nbr_b, m_w1, m_b1, m_w2, m_g, m_b, lfp_w, lfp_g,
           lfp_b, mm_w1, mm_b1, mm_w2, mm_g, mm_b, pp_g, pp_b, pp_w):
    height = xyz[..., 1:2] / 10.0
    height = height - height.min(axis=1, keepdims=True)
    g8 = jnp.concatenate(
        [xyz, x, height, jnp.zeros_like(height)], axis=-1).reshape(ROWS * 8)
    xyz8 = jnp.pad(xyz.reshape(ROWS, 3), ((0, 0), (0, 5)))

    knng, nbr = _sc_topk(pwd.reshape(ROWS, N), g8)
    idx3 = knng.reshape(NW, NG, GR)
    nbr = nbr.reshape(ROWS, K * 8)

    w1p = jnp.pad(ne_w1, ((0, 1), (0, 0)))
    h = _nbr_mlp(nbr, xyz8, w1p, ne_b1[None], ne_w2, ne_b2[None], ne_w3)

    xx, y, ybf = _bn_mlp_y(h, nbr_g[None], nbr_b[None], m_w1, m_b1[None],
                           m_w2, m_g[None], m_b[None], lfp_w[0])

    def gmax_of(ybf):
        ybf32 = lax.bitcast_convert_type(
            ybf.reshape(ROWS, DIM // 2, 2), jnp.int32)
        return lax.bitcast_convert_type(
            _sc_gather_max(ybf32, idx3), jnp.bfloat16).reshape(ROWS, DIM)

    gmax = gmax_of(ybf)
    xx, y, ybf = _comb_y(xx, gmax, y, lfp_g[0][None], lfp_b[0][None],
                         lfp_w[1])
    gmax = gmax_of(ybf)
    xx, y, ybf = _comb_mm_y(xx, gmax, y, lfp_g[1][None], lfp_b[1][None],
                            mm_w1[0], mm_b1[0][None], mm_w2[0], mm_g[0][None],
                            mm_b[0][None], lfp_w[2])
    gmax = gmax_of(ybf)
    xx, y, ybf = _comb_y(xx, gmax, y, lfp_g[2][None], lfp_b[2][None],
                         lfp_w[3])
    gmax = gmax_of(ybf)
    return _comb_mm_fin(xx, gmax, y, lfp_g[3][None], lfp_b[3][None],
                        mm_w1[1], mm_b1[1][None], mm_w2[1], mm_g[1][None],
                        mm_b[1][None], pp_g[None], pp_b[None], pp_w)


# speedup vs baseline: 17.5587x; 1.0025x over previous
import math

import jax
import jax.numpy as jnp
from jax import lax
from jax.experimental import pallas as pl
from jax.experimental.pallas import tpu as pltpu
from jax.experimental.pallas import tpu_sc as plsc

B, N, K, DIM, HEAD_DIM = 4, 2048, 24, 256, 256
EPS = 1e-5
BN_SCALE = 1.0 / math.sqrt(1.0 + EPS)
ROWS = B * N

NC, NS = 2, 16
NW = NC * NS
PPW = ROWS // NW
PG = 4
NG = PPW // PG
GR = PG * K


def _gmax_body(tab_hbm, idx_hbm, out_hbm, idx_v, rows0, rows1, out_v, sem0,
               sem1):
    wid = lax.axis_index("s") * NC + lax.axis_index("c")
    pltpu.sync_copy(idx_hbm.at[wid], idx_v)
    bufs = (rows0, rows1)
    sems = (sem0, sem1)

    def issue(g, slot):
        pltpu.async_copy(tab_hbm.at[idx_v.at[g]], bufs[slot], sems[slot])

    def wait(slot):
        pltpu.make_async_copy(tab_hbm.at[pl.ds(0, GR)], bufs[slot],
                              sems[slot]).wait()

    def compute(g, slot):
        rows = bufs[slot]

        def p_body(p, _):
            r0 = p * K

            def t_body(t, _):
                c0 = pl.multiple_of(t * 16, 16)
                acc = plsc.bitcast(rows[r0, pl.ds(c0, 16)], jnp.bfloat16)
                for j in range(1, K):
                    acc = jnp.maximum(
                        acc, plsc.bitcast(rows[r0 + j, pl.ds(c0, 16)],
                                          jnp.bfloat16))
                out_v[g * PG + p, pl.ds(c0, 16)] = plsc.bitcast(
                    acc, jnp.int32)
                return 0

            lax.fori_loop(0, DIM // 32, t_body, 0)
            return 0

        lax.fori_loop(0, PG, p_body, 0)

    issue(0, 0)

    def g_body(g2, _):
        g = g2 * 2
        wait(0)
        issue(g + 1, 1)
        compute(g, 0)
        wait(1)

        @pl.when(g + 2 < NG)
        def _():
            issue(g + 2, 0)

        compute(g + 1, 1)
        return 0

    lax.fori_loop(0, NG // 2, g_body, 0)
    pltpu.sync_copy(out_v, out_hbm.at[pl.ds(wid * PPW, PPW)])


def _sc_gather_max(tab, idx3):
    mesh = plsc.VectorSubcoreMesh(core_axis_name="c", subcore_axis_name="s")
    f = pl.kernel(
        _gmax_body,
        mesh=mesh,
        out_type=jax.ShapeDtypeStruct((ROWS, DIM // 2), jnp.int32),
        scratch_types=[
            pltpu.VMEM((NG, GR), jnp.int32),
            pltpu.VMEM((GR, DIM // 2), jnp.int32),
            pltpu.VMEM((GR, DIM // 2), jnp.int32),
            pltpu.VMEM((PPW, DIM // 2), jnp.int32),
            pltpu.SemaphoreType.DMA,
            pltpu.SemaphoreType.DMA,
        ],
        compiler_params=pltpu.CompilerParams(needs_layout_passes=False),
    )
    return f(tab, idx3)


_TKW = 4
_INF = float('inf')


def _merge16_kv(ka, va, kb, vb):
    krb = lax.rev(kb, (0,))
    vrb = lax.rev(vb, (0,))
    cmp = ka <= krb
    lok = jnp.minimum(ka, krb)
    hik = jnp.maximum(ka, krb)
    lov = jnp.where(cmp, va, vrb)
    hiv = jnp.where(cmp, vrb, va)
    lok, lov = plsc.sort_key_val(lok, lov)
    hik, hiv = plsc.sort_key_val(hik, hiv)
    return lok, lov, hik, hiv


def _topk_body(pwd_hbm, g8_hbm, out_hbm, nbr_hbm, buf0, buf1, cval, cidx,
               outv, tab_v, nbrw, sem0, sem1):
    wid = lax.axis_index("s") * NC + lax.axis_index("c")
    row0 = wid * PPW
    boff = (wid // (NW // B)) * N
    ii = jnp.arange(16, dtype=jnp.int32)
    iif = ii.astype(jnp.float32)
    bufs = (buf0, buf1)
    sems = (sem0, sem1)
    nwin = PPW // _TKW
    pltpu.sync_copy(g8_hbm, tab_v)

    def issue(g, slot):
        pltpu.async_copy(pwd_hbm.at[pl.ds(row0 + g * _TKW, _TKW)],
                         bufs[slot], sems[slot])

    def wait(slot):
        pltpu.make_async_copy(pwd_hbm.at[pl.ds(0, _TKW)], bufs[slot],
                              sems[slot]).wait()

    def window(g, slot):
        buf = bufs[slot]

        def row_body(r, _):
            def pa(c, carry):
                m1, m2 = carry
                for u in range(2):
                    v = buf[r, pl.ds(pl.multiple_of(c * 32 + u * 16, 16), 16)]
                    nm1 = jnp.minimum(m1, v)
                    m2 = jnp.minimum(m2, jnp.maximum(m1, v))
                    m1 = nm1
                return m1, m2

            m1, m2 = lax.fori_loop(
                0, 64, pa, (jnp.full((16,), _INF), jnp.full((16,), _INF)))
            s1, _ = plsc.sort_key_val(m1, ii)
            s2, _ = plsc.sort_key_val(m2, ii)
            _, _, hik, _ = _merge16_kv(s1, iif, s2, iif)
            tau = jnp.max(jnp.where(ii == 7, hik, -_INF))
            tauv = jnp.full((16,), tau)

            def pb(c, offv):
                for u in range(2):
                    cb = pl.multiple_of(c * 32 + u * 16, 16)
                    v = buf[r, pl.ds(cb, 16)]
                    msk = v <= tauv
                    pos = offv + plsc.cumsum(msk.astype(jnp.int32)) - 1
                    plsc.store_scatter(cval, [pos], v, mask=msk)
                    plsc.store_scatter(cidx, [pos], ii + cb, mask=msk)
                    offv = offv + plsc.all_reduce_population_count(msk)
                return offv

            offv = lax.fori_loop(0, 64, pb, jnp.zeros((16,), jnp.int32))
            cnt = jnp.max(offv)

            nchunk = (cnt + 15) // 16
            rem0 = cnt - (nchunk - 1) * 16

            def sel(ci, carry):
                r0k, r0v, r1k, r1v = carry
                cb = ci * 16
                ck = plsc.load_gather(cval, [cb + ii])
                cv = plsc.load_gather(cidx, [cb + ii])
                nvalid = jnp.where(ci == nchunk - 1, rem0, 16)
                ck = jnp.where(ii < nvalid, ck, _INF)
                ck, cv = plsc.sort_key_val(ck, cv)
                crk = lax.rev(ck, (0,))
                crv = lax.rev(cv, (0,))
                cmp = r1k <= crk
                n1k = jnp.minimum(r1k, crk)
                n1v = jnp.where(cmp, r1v, crv)
                n1k, n1v = plsc.sort_key_val(n1k, n1v)
                return _merge16_kv(r0k, r0v, n1k, n1v)

            init = (jnp.full((16,), _INF), ii, jnp.full((16,), _INF), ii)
            r0k, r0v, r1k, r1v = lax.fori_loop(0, nchunk, sel, init)

            p = g * _TKW + r
            gid0 = r0v + boff
            gid1 = r1v + boff
            prow = p // PG
            poff = (p % PG) * K
            outv[prow, pl.ds(pl.multiple_of(poff, 8), 16)] = gid0
            rsplat = jnp.full((16,), prow, jnp.int32)
            plsc.store_scatter(outv, [rsplat, poff + 16 + ii], gid1,
                               mask=ii < 8)

            a1 = gid0 * 8
            a2 = gid1 * 8
            o1 = ii * 8
            m8 = ii < 8
            rsp = jnp.full((16,), r, jnp.int32)
            for c in range(8):
                g1 = plsc.load_gather(tab_v, [a1 + c])
                g2 = plsc.load_gather(tab_v, [a2 + c], mask=m8)
                plsc.store_scatter(nbrw, [rsp, o1 + c], g1)
                plsc.store_scatter(nbrw, [rsp, o1 + 128 + c], g2, mask=m8)
            return 0

        lax.fori_loop(0, _TKW, row_body, 0)
        pltpu.sync_copy(
            nbrw, nbr_hbm.at[pl.ds(row0 + g * _TKW, _TKW)])

    issue(0, 0)

    def w_body(g2, _):
        g = g2 * 2
        wait(0)
        issue(g + 1, 1)
        window(g, 0)
        wait(1)

        @pl.when(g + 2 < nwin)
        def _():
            issue(g + 2, 0)

        window(g + 1, 1)
        return 0

    lax.fori_loop(0, nwin // 2, w_body, 0)
    pltpu.sync_copy(outv, out_hbm.at[wid])


def _sc_topk(pwd2, g8f):
    mesh = plsc.VectorSubcoreMesh(core_axis_name="c", subcore_axis_name="s")
    f = pl.kernel(
        _topk_body,
        mesh=mesh,
        out_type=(jax.ShapeDtypeStruct((NW, NG, GR), jnp.int32),
                  jax.ShapeDtypeStruct((ROWS, K * 8), jnp.float32)),
        scratch_types=[
            pltpu.VMEM((_TKW, N), jnp.float32),
            pltpu.VMEM((_TKW, N), jnp.float32),
            pltpu.VMEM((N,), jnp.float32),
            pltpu.VMEM((N,), jnp.int32),
            pltpu.VMEM((NG, GR), jnp.int32),
            pltpu.VMEM((ROWS * 8,), jnp.float32),
            pltpu.VMEM((_TKW, K * 8), jnp.float32),
            pltpu.SemaphoreType.DMA,
            pltpu.SemaphoreType.DMA,
        ],
        compiler_params=pltpu.CompilerParams(needs_layout_passes=False),
    )
    return f(pwd2, g8f)


def _gelu(x):
    return 0.5 * x * (1.0 + lax.erf(x * (1.0 / math.sqrt(2.0))))


def _nbr_mlp_kernel(nbr_ref, c8_ref, w1_ref, b1_ref, w2_ref, b2_ref, w3_ref,
                    out_ref):
    c8 = c8_ref[...]
    w1 = w1_ref[...]
    w2 = w2_ref[...]
    w3 = w3_ref[...].astype(jnp.bfloat16)
    hmax = None
    for j in range(K):
        x = nbr_ref[:, j * 8:(j + 1) * 8] - c8
        h = _gelu(jnp.dot(x, w1, preferred_element_type=jnp.float32)
                  * BN_SCALE + b1_ref[...])
        h = _gelu(jnp.dot(h, w2, preferred_element_type=jnp.float32)
                  * BN_SCALE + b2_ref[...])
        h = jnp.dot(h.astype(jnp.bfloat16), w3,
                    preferred_element_type=jnp.float32)
        hmax = h if hmax is None else jnp.maximum(hmax, h)
    out_ref[...] = hmax


def _nbr_mlp(nbr, c8, w1, b1, w2, b2, w3, p_blk=512):
    return pl.pallas_call(
        _nbr_mlp_kernel,
        grid=(ROWS // p_blk,),
        in_specs=[
            pl.BlockSpec((p_blk, K * 8), lambda i: (i, 0)),
            pl.BlockSpec((p_blk, 8), lambda i: (i, 0)),
            pl.BlockSpec((8, 16), lambda i: (0, 0)),
            pl.BlockSpec((1, 16), lambda i: (0, 0)),
            pl.BlockSpec((16, 32), lambda i: (0, 0)),
            pl.BlockSpec((1, 32), lambda i: (0, 0)),
            pl.BlockSpec((32, DIM), lambda i: (0, 0)),
        ],
        out_specs=pl.BlockSpec((p_blk, DIM), lambda i: (i, 0)),
        out_shape=jax.ShapeDtypeStruct((ROWS, DIM), jnp.float32),
    )(nbr, c8, w1, b1, w2, b2, w3)


_ROW_SPEC = pl.BlockSpec((1024, DIM), lambda i: (i, 0))
_VEC_SPEC = pl.BlockSpec((1, DIM), lambda i: (0, 0))
_VEC2_SPEC = pl.BlockSpec((1, 2 * DIM), lambda i: (0, 0))
_W_SPEC = pl.BlockSpec((DIM, DIM), lambda i: (0, 0))
_W12_SPEC = pl.BlockSpec((DIM, 2 * DIM), lambda i: (0, 0))
_W21_SPEC = pl.BlockSpec((2 * DIM, DIM), lambda i: (0, 0))
_YOUT = [jax.ShapeDtypeStruct((ROWS, DIM), jnp.float32),
         jax.ShapeDtypeStruct((ROWS, DIM), jnp.float32),
         jax.ShapeDtypeStruct((ROWS, DIM), jnp.bfloat16)]
_YOUT_SPEC = [_ROW_SPEC, _ROW_SPEC, _ROW_SPEC]


def _mlp_block(xx, w1, b1, w2, g, b):
    t = _gelu(jnp.dot(xx, w1, preferred_element_type=jnp.float32) + b1)
    t = jnp.dot(t, w2, preferred_element_type=jnp.float32)
    return xx + t * BN_SCALE * g + b


def _emit_y(xx, w_n, xx_ref, y_ref, yb_ref):
    y = jnp.dot(xx, w_n, preferred_element_type=jnp.float32)
    xx_ref[...] = xx
    y_ref[...] = y
    yb_ref[...] = y.astype(jnp.bfloat16)


def _bn_mlp_y_kernel(h_ref, g0_ref, b0_ref, w1_ref, b1_ref, w2_ref, g_ref,
                     b_ref, wn_ref, xx_ref, y_ref, yb_ref):
    xx = h_ref[...] * BN_SCALE * g0_ref[...] + b0_ref[...]
    xx = _mlp_block(xx, w1_ref[...], b1_ref[...], w2_ref[...], g_ref[...],
                    b_ref[...])
    _emit_y(xx, wn_ref[...], xx_ref, y_ref, yb_ref)


def _bn_mlp_y(h, g0, b0, w1, b1, w2, g, b, wn):
    return pl.pallas_call(
        _bn_mlp_y_kernel,
        grid=(8,),
        in_specs=[_ROW_SPEC, _VEC_SPEC, _VEC_SPEC, _W12_SPEC, _VEC2_SPEC,
                  _W21_SPEC, _VEC_SPEC, _VEC_SPEC, _W_SPEC],
        out_specs=_YOUT_SPEC,
        out_shape=_YOUT,
    )(h, g0, b0, w1, b1, w2, g, b, wn)


def _combine(xx_ref, gmax_ref, y_ref, g_ref, b_ref):
    edge = gmax_ref[...].astype(jnp.float32) - y_ref[...]
    return xx_ref[...] + edge * BN_SCALE * g_ref[...] + b_ref[...]


def _comb_y_kernel(xx_ref, gmax_ref, y_ref, g_ref, b_ref, wn_ref, xxo_ref,
                   yo_ref, ybo_ref):
    xx = _combine(xx_ref, gmax_ref, y_ref, g_ref, b_ref)
    _emit_y(xx, wn_ref[...], xxo_ref, yo_ref, ybo_ref)


def _comb_y(xx, gmax, y, g, b, wn):
    return pl.pallas_call(
        _comb_y_kernel,
        grid=(8,),
        in_specs=[_ROW_SPEC, _ROW_SPEC, _ROW_SPEC, _VEC_SPEC, _VEC_SPEC,
                  _W_SPEC],
        out_specs=_YOUT_SPEC,
        out_shape=_YOUT,
    )(xx, gmax, y, g, b, wn)


def _comb_mm_y_kernel(xx_ref, gmax_ref, y_ref, g_ref, b_ref, w1_ref, b1_ref,
                      w2_ref, mg_ref, mb_ref, wn_ref, xxo_ref, yo_ref,
                      ybo_ref):
    xx = _combine(xx_ref, gmax_ref, y_ref, g_ref, b_ref)
    xx = _mlp_block(xx, w1_ref[...], b1_ref[...], w2_ref[...], mg_ref[...],
                    mb_ref[...])
    _emit_y(xx, wn_ref[...], xxo_ref, yo_ref, ybo_ref)


def _comb_mm_y(xx, gmax, y, g, b, w1, b1, w2, mg, mb, wn):
    return pl.pallas_call(
        _comb_mm_y_kernel,
        grid=(8,),
        in_specs=[_ROW_SPEC, _ROW_SPEC, _ROW_SPEC, _VEC_SPEC, _VEC_SPEC,
                  _W12_SPEC, _VEC2_SPEC, _W21_SPEC, _VEC_SPEC, _VEC_SPEC,
                  _W_SPEC],
        out_specs=_YOUT_SPEC,
        out_shape=_YOUT,
    )(xx, gmax, y, g, b, w1, b1, w2, mg, mb, wn)


def _comb_mm_fin_kernel(xx_ref, gmax_ref, y_ref, g_ref, b_ref, w1_ref,
                        b1_ref, w2_ref, mg_ref, mb_ref, pg_ref, pb_ref,
                        pw_ref, out_ref):
    xx = _combine(xx_ref, gmax_ref, y_ref, g_ref, b_ref)
    xx = _mlp_block(xx, w1_ref[...], b1_ref[...], w2_ref[...], mg_ref[...],
                    mb_ref[...])
    t = xx * BN_SCALE * pg_ref[...] + pb_ref[...]
    out_ref[...] = jnp.dot(t, pw_ref[...], preferred_element_type=jnp.float32)


def _comb_mm_fin(xx, gmax, y, g, b, w1, b1, w2, mg, mb, pg, pb, pw):
    return pl.pallas_call(
        _comb_mm_fin_kernel,
        grid=(8,),
        in_specs=[_ROW_SPEC, _ROW_SPEC, _ROW_SPEC, _VEC_SPEC, _VEC_SPEC,
                  _W12_SPEC, _VEC2_SPEC, _W21_SPEC, _VEC_SPEC, _VEC_SPEC,
                  _VEC_SPEC, _VEC_SPEC,
                  pl.BlockSpec((DIM, HEAD_DIM), lambda i: (0, 0))],
        out_specs=pl.BlockSpec((1024, HEAD_DIM), lambda i: (i, 0)),
        out_shape=jax.ShapeDtypeStruct((ROWS, HEAD_DIM), jnp.float32),
    )(xx, gmax, y, g, b, w1, b1, w2, mg, mb, pg, pb, pw)


def kernel(x, xyz, prev_knn, pwd, ne_w1, ne_g1, ne_b1, ne_w2, ne_g2, ne_b2,
           ne_w3, nbr_g, nbr_b, m_w1, m_b1, m_w2, m_g, m_b, lfp_w, lfp_g,
           lfp_b, mm_w1, mm_b1, mm_w2, mm_g, mm_b, pp_g, pp_b, pp_w):
    height = xyz[..., 1:2] / 10.0
    height = height - height.min(axis=1, keepdims=True)
    g8 = jnp.concatenate(
        [xyz, x, height, jnp.zeros_like(height)], axis=-1).reshape(ROWS * 8)
    xyz8 = jnp.pad(xyz.reshape(ROWS, 3), ((0, 0), (0, 5)))

    idx3, nbr = _sc_topk(pwd.reshape(ROWS, N), g8)

    w1p = jnp.pad(ne_w1, ((0, 1), (0, 0)))
    h = _nbr_mlp(nbr, xyz8, w1p, ne_b1[None], ne_w2, ne_b2[None], ne_w3)

    xx, y, ybf = _bn_mlp_y(h, nbr_g[None], nbr_b[None], m_w1, m_b1[None],
                           m_w2, m_g[None], m_b[None], lfp_w[0])

    def gmax_of(ybf):
        ybf32 = lax.bitcast_convert_type(
            ybf.reshape(ROWS, DIM // 2, 2), jnp.int32)
        return lax.bitcast_convert_type(
            _sc_gather_max(ybf32, idx3), jnp.bfloat16).reshape(ROWS, DIM)

    gmax = gmax_of(ybf)
    xx, y, ybf = _comb_y(xx, gmax, y, lfp_g[0][None], lfp_b[0][None],
                         lfp_w[1])
    gmax = gmax_of(ybf)
    xx, y, ybf = _comb_mm_y(xx, gmax, y, lfp_g[1][None], lfp_b[1][None],
                            mm_w1[0], mm_b1[0][None], mm_w2[0], mm_g[0][None],
                            mm_b[0][None], lfp_w[2])
    gmax = gmax_of(ybf)
    xx, y, ybf = _comb_y(xx, gmax, y, lfp_g[2][None], lfp_b[2][None],
                         lfp_w[3])
    gmax = gmax_of(ybf)
    return _comb_mm_fin(xx, gmax, y, lfp_g[3][None], lfp_b[3][None],
                        mm_w1[1], mm_b1[1][None], mm_w2[1], mm_g[1][None],
                        mm_b[1][None], pp_g[None], pp_b[None], pp_w)
